# Initial kernel scaffold; baseline (speedup 1.0000x reference)
import jax
import jax.numpy as jnp
from jax.experimental import pallas as pl


def kernel(node_features, edge_index, current_focal_leaf, branch_child, time_value, is_root, W1, b1, W2, b2, Wh1, bh1, Wh2, bh2, Wh3, bh3):
    raise NotImplementedError("write your pallas kernel here")


# speedup vs baseline: 1.6609x; 1.6609x over previous
import functools

import jax
import jax.numpy as jnp
from jax import lax
from jax.experimental import pallas as pl
from jax.experimental.pallas import tpu as pltpu
from jax.experimental.pallas import tpu_sc as plsc

N = 100001
F_IN = 128
H = 64
A = 100000

NC, NS = 2, 16
NW = NC * NS
BN = 512
PAD = 100352
B_PER_W = PAD // NW
GW = 112
CHUNKS = B_PER_W // GW


def _isd_block(ei):
    mask = (ei >= 0).astype(jnp.float32)
    deg = jnp.sum(mask, axis=1, keepdims=True) + 1.0
    return lax.rsqrt(deg)


def _mm1_body(x_ref, ei_ref, w_ref, b_ref, y_ref):
    i = pl.program_id(0)
    rows = i * BN + lax.broadcasted_iota(jnp.int32, (BN, 1), 0)
    isd = _isd_block(ei_ref[...])
    xw = jnp.dot(x_ref[...], w_ref[...], preferred_element_type=jnp.float32)
    y = isd * (xw + b_ref[...])
    y_ref[...] = jnp.where(rows < N, y, 0.0)


def _mm2_body(y_ref, a_ref, ei_ref, w_ref, b_ref, o_ref):
    i = pl.program_id(0)
    rows = i * BN + lax.broadcasted_iota(jnp.int32, (BN, 1), 0)
    isd = _isd_block(ei_ref[...])
    h = jnp.maximum(isd * (y_ref[...] + a_ref[...]), 0.0)
    xw = jnp.dot(h, w_ref[...], preferred_element_type=jnp.float32)
    y2 = isd * (xw + b_ref[...])
    o_ref[...] = jnp.where(rows < N, y2, 0.0)


def _emb_body(y_ref, a_ref, ei_ref, o_ref):
    isd = _isd_block(ei_ref[...])
    o_ref[...] = isd * (y_ref[...] + a_ref[...])


def _elu(x):
    return jnp.where(x > 0, x, jnp.exp(x) - 1.0)


def _mlp_body(ht_ref, tv_ref, ir_ref, hf_ref, w1_ref, b1_ref, w2_ref, b2_ref,
              w3_ref, b3_ref, ef_ref, lg_ref):
    ht = ht_ref[...]
    hfb = jnp.broadcast_to(hf_ref[...], ht.shape)
    ad = jnp.abs(hfb - ht)
    pr = hfb * ht
    t = tv_ref[...] / jnp.float32(1.0 + 1e-8)
    ef = jnp.concatenate([hfb, ht, ad, pr, t, ir_ref[...]], axis=1)
    ef_ref[...] = ef
    z = _elu(jnp.dot(ef, w1_ref[...], preferred_element_type=jnp.float32)
             + b1_ref[...])
    z = _elu(jnp.dot(z, w2_ref[...], preferred_element_type=jnp.float32)
             + b2_ref[...])
    lg_ref[...] = (jnp.dot(z, w3_ref[...], preferred_element_type=jnp.float32)
                   + b3_ref[...])


def _softmax_body(x_ref, o_ref):
    x = x_ref[...]
    m = jnp.max(x)
    e = jnp.exp(x - m)
    o_ref[...] = e / jnp.sum(e)


def _sc_mesh():
    return plsc.VectorSubcoreMesh(core_axis_name="c", subcore_axis_name="s",
                                  num_cores=NC, num_subcores=NS)


_SC_PARAMS = pltpu.CompilerParams(use_tc_tiling_on_sc=False)


def _sc_agg(y_tbl, i0h, i1h, i2h):
    @functools.partial(
        pl.kernel,
        out_type=jax.ShapeDtypeStruct((PAD, H), jnp.float32),
        mesh=_sc_mesh(),
        compiler_params=_SC_PARAMS,
        scratch_types=[
            pltpu.VMEM((GW,), jnp.int32),
            pltpu.VMEM((GW,), jnp.int32),
            pltpu.VMEM((GW,), jnp.int32),
            pltpu.VMEM((GW, H), jnp.float32),
            pltpu.VMEM((GW, H), jnp.float32),
            pltpu.VMEM((GW, H), jnp.float32),
            pltpu.SemaphoreType.DMA,
        ],
    )
    def k(y_hbm, i0_hbm, i1_hbm, i2_hbm, out_hbm, iv0, iv1, iv2, g0, g1, g2,
          sem):
        wid = lax.axis_index("s") * NC + lax.axis_index("c")
        base = wid * B_PER_W

        @pl.loop(0, CHUNKS)
        def _(c):
            off = base + c * GW
            pltpu.sync_copy(i0_hbm.at[pl.ds(off, GW)], iv0)
            pltpu.sync_copy(i1_hbm.at[pl.ds(off, GW)], iv1)
            pltpu.sync_copy(i2_hbm.at[pl.ds(off, GW)], iv2)
            d0 = pltpu.async_copy(y_hbm.at[iv0], g0, sem)
            d1 = pltpu.async_copy(y_hbm.at[iv1], g1, sem)
            d2 = pltpu.async_copy(y_hbm.at[iv2], g2, sem)
            d0.wait()
            d1.wait()
            d2.wait()

            @pl.loop(0, GW)
            def _(r):
                @pl.loop(0, H, step=16)
                def _(l):
                    sl = pl.ds(l, 16)
                    g0.at[r, sl][...] = (g0.at[r, sl][...]
                                         + g1.at[r, sl][...]
                                         + g2.at[r, sl][...])

            pltpu.sync_copy(g0, out_hbm.at[pl.ds(off, GW)])

    return k(y_tbl, i0h, i1h, i2h)


def _sc_gather(tbl, idx):
    @functools.partial(
        pl.kernel,
        out_type=jax.ShapeDtypeStruct((PAD, H), jnp.float32),
        mesh=_sc_mesh(),
        compiler_params=_SC_PARAMS,
        scratch_types=[
            pltpu.VMEM((GW,), jnp.int32),
            pltpu.VMEM((GW, H), jnp.float32),
            pltpu.SemaphoreType.DMA,
        ],
    )
    def k(t_hbm, i_hbm, out_hbm, iv, rows, sem):
        wid = lax.axis_index("s") * NC + lax.axis_index("c")
        base = wid * B_PER_W

        @pl.loop(0, CHUNKS)
        def _(c):
            off = base + c * GW
            pltpu.sync_copy(i_hbm.at[pl.ds(off, GW)], iv)
            pltpu.async_copy(t_hbm.at[iv], rows, sem).wait()
            pltpu.sync_copy(rows, out_hbm.at[pl.ds(off, GW)])

    return k(tbl, idx)


def kernel(node_features, edge_index, current_focal_leaf, branch_child,
           time_value, is_root, W1, b1, W2, b2, Wh1, bh1, Wh2, bh2, Wh3, bh3):
    f32 = jnp.float32

    safe = jnp.where(edge_index < 0, jnp.int32(N), edge_index)
    safe = jnp.pad(safe, ((0, PAD - N), (0, 0)), constant_values=N)
    i0, i1, i2 = safe[:, 0], safe[:, 1], safe[:, 2]
    bc = jnp.pad(branch_child, (0, PAD - A))

    b1r = b1.reshape(1, H)
    b2r = b2.reshape(1, H)
    bh1r = bh1.reshape(1, H)
    bh2r = bh2.reshape(1, H)
    bh3r = bh3.reshape(1, 1)

    g_rows = PAD // BN
    g_n = -(-N // BN)
    g_a = -(-A // BN)

    wspec = lambda shape: pl.BlockSpec(shape, lambda i: (0, 0))

    y1 = pl.pallas_call(
        _mm1_body,
        grid=(g_rows,),
        in_specs=[
            pl.BlockSpec((BN, F_IN), lambda i: (i, 0)),
            pl.BlockSpec((BN, 3), lambda i: (i, 0)),
            wspec((F_IN, H)),
            wspec((1, H)),
        ],
        out_specs=pl.BlockSpec((BN, H), lambda i: (i, 0)),
        out_shape=jax.ShapeDtypeStruct((PAD, H), f32),
    )(node_features, edge_index, W1, b1r)

    agg1 = _sc_agg(y1, i0, i1, i2)

    y2 = pl.pallas_call(
        _mm2_body,
        grid=(g_rows,),
        in_specs=[
            pl.BlockSpec((BN, H), lambda i: (i, 0)),
            pl.BlockSpec((BN, H), lambda i: (i, 0)),
            pl.BlockSpec((BN, 3), lambda i: (i, 0)),
            wspec((H, H)),
            wspec((1, H)),
        ],
        out_specs=pl.BlockSpec((BN, H), lambda i: (i, 0)),
        out_shape=jax.ShapeDtypeStruct((PAD, H), f32),
    )(y1, agg1, edge_index, W2, b2r)

    agg2 = _sc_agg(y2, i0, i1, i2)

    node_emb = pl.pallas_call(
        _emb_body,
        grid=(g_n,),
        in_specs=[
            pl.BlockSpec((BN, H), lambda i: (i, 0)),
            pl.BlockSpec((BN, H), lambda i: (i, 0)),
            pl.BlockSpec((BN, 3), lambda i: (i, 0)),
        ],
        out_specs=pl.BlockSpec((BN, H), lambda i: (i, 0)),
        out_shape=jax.ShapeDtypeStruct((N, H), f32),
    )(y2, agg2, edge_index)

    h_target = _sc_gather(node_emb, bc)

    h_focal = lax.dynamic_slice(node_emb, (N - 1, 0), (1, H))

    ef, lg = pl.pallas_call(
        _mlp_body,
        grid=(g_a,),
        in_specs=[
            pl.BlockSpec((BN, H), lambda i: (i, 0)),
            pl.BlockSpec((BN, 1), lambda i: (i, 0)),
            pl.BlockSpec((BN, 1), lambda i: (i, 0)),
            wspec((1, H)),
            wspec((4 * H + 2, H)),
            wspec((1, H)),
            wspec((H, H)),
            wspec((1, H)),
            wspec((H, 1)),
            wspec((1, 1)),
        ],
        out_specs=[
            pl.BlockSpec((BN, 4 * H + 2), lambda i: (i, 0)),
            pl.BlockSpec((BN, 1), lambda i: (i, 0)),
        ],
        out_shape=[
            jax.ShapeDtypeStruct((A, 4 * H + 2), f32),
            jax.ShapeDtypeStruct((A, 1), f32),
        ],
    )(h_target, time_value.reshape(A, 1), is_root.reshape(A, 1), h_focal,
      Wh1, bh1r, Wh2, bh2r, Wh3, bh3r)

    probs = pl.pallas_call(
        _softmax_body,
        grid=(1,),
        in_specs=[pl.BlockSpec((8, A // 8), lambda i: (0, 0))],
        out_specs=pl.BlockSpec((8, A // 8), lambda i: (0, 0)),
        out_shape=jax.ShapeDtypeStruct((8, A // 8), f32),
    )(lg.reshape(8, A // 8))

    action_logits = lg.reshape(A)
    action_probs = probs.reshape(A)
    leaf_feature = jax.nn.one_hot(current_focal_leaf, F_IN, dtype=f32)
    return (action_logits, action_probs, ef, node_emb, leaf_feature)


# speedup vs baseline: 1.6633x; 1.0014x over previous
import functools

import jax
import jax.numpy as jnp
from jax import lax
from jax.experimental import pallas as pl
from jax.experimental.pallas import tpu as pltpu
from jax.experimental.pallas import tpu_sc as plsc

N = 100001
F_IN = 128
H = 64
A = 100000

NC, NS = 2, 16
NW = NC * NS
BN = 512
PAD = 100352
B_PER_W = PAD // NW
GW = 112
CHUNKS = B_PER_W // GW


def _isd_block(ei):
    mask = (ei >= 0).astype(jnp.float32)
    deg = jnp.sum(mask, axis=1, keepdims=True) + 1.0
    return lax.rsqrt(deg)


def _mm1_body(x_ref, ei_ref, w_ref, b_ref, y_ref):
    i = pl.program_id(0)
    rows = i * BN + lax.broadcasted_iota(jnp.int32, (BN, 1), 0)
    isd = _isd_block(ei_ref[...])
    xw = jnp.dot(x_ref[...], w_ref[...], preferred_element_type=jnp.float32)
    y = isd * (xw + b_ref[...])
    y_ref[...] = jnp.where(rows < N, y, 0.0)


def _mm2_body(y_ref, a_ref, ei_ref, w_ref, b_ref, o_ref):
    i = pl.program_id(0)
    rows = i * BN + lax.broadcasted_iota(jnp.int32, (BN, 1), 0)
    isd = _isd_block(ei_ref[...])
    h = jnp.maximum(isd * (y_ref[...] + a_ref[...]), 0.0)
    xw = jnp.dot(h, w_ref[...], preferred_element_type=jnp.float32)
    y2 = isd * (xw + b_ref[...])
    o_ref[...] = jnp.where(rows < N, y2, 0.0)


def _emb_body(y_ref, a_ref, ei_ref, o_ref):
    isd = _isd_block(ei_ref[...])
    o_ref[...] = isd * (y_ref[...] + a_ref[...])


def _elu(x):
    return jnp.where(x > 0, x, jnp.exp(x) - 1.0)


def _mlp_body(ht_ref, tv_ref, ir_ref, hf_ref, w1_ref, b1_ref, w2_ref, b2_ref,
              w3_ref, b3_ref, ef_ref, lg_ref):
    ht = ht_ref[...]
    hfb = jnp.broadcast_to(hf_ref[...], ht.shape)
    ad = jnp.abs(hfb - ht)
    pr = hfb * ht
    t = tv_ref[...] / jnp.float32(1.0 + 1e-8)
    ef = jnp.concatenate([hfb, ht, ad, pr, t, ir_ref[...]], axis=1)
    ef_ref[...] = ef
    z = _elu(jnp.dot(ef, w1_ref[...], preferred_element_type=jnp.float32)
             + b1_ref[...])
    z = _elu(jnp.dot(z, w2_ref[...], preferred_element_type=jnp.float32)
             + b2_ref[...])
    lg_ref[...] = (jnp.dot(z, w3_ref[...], preferred_element_type=jnp.float32)
                   + b3_ref[...])


def _softmax_body(x_ref, o_ref):
    x = x_ref[...]
    m = jnp.max(x)
    e = jnp.exp(x - m)
    o_ref[...] = e / jnp.sum(e)


def _sc_mesh():
    return plsc.VectorSubcoreMesh(core_axis_name="c", subcore_axis_name="s",
                                  num_cores=NC, num_subcores=NS)


_SC_PARAMS = pltpu.CompilerParams(use_tc_tiling_on_sc=False)


def _sc_agg(y_tbl, i0h, i1h, i2h):
    @functools.partial(
        pl.kernel,
        out_type=jax.ShapeDtypeStruct((PAD, H), jnp.float32),
        mesh=_sc_mesh(),
        compiler_params=_SC_PARAMS,
        scratch_types=[
            pltpu.VMEM((B_PER_W,), jnp.int32),
            pltpu.VMEM((B_PER_W,), jnp.int32),
            pltpu.VMEM((B_PER_W,), jnp.int32),
            pltpu.VMEM((GW, H), jnp.float32),
            pltpu.VMEM((GW, H), jnp.float32),
            pltpu.VMEM((GW, H), jnp.float32),
            pltpu.VMEM((GW, H), jnp.float32),
            pltpu.VMEM((GW, H), jnp.float32),
            pltpu.VMEM((GW, H), jnp.float32),
            pltpu.VMEM((GW, H), jnp.float32),
            pltpu.VMEM((GW, H), jnp.float32),
            pltpu.SemaphoreType.DMA,
            pltpu.SemaphoreType.DMA,
            pltpu.SemaphoreType.DMA,
            pltpu.SemaphoreType.DMA,
            pltpu.SemaphoreType.DMA,
        ],
    )
    def k(y_hbm, i0_hbm, i1_hbm, i2_hbm, out_hbm,
          iv0, iv1, iv2, ga0, ga1, ga2, gb0, gb1, gb2, aca, acb,
          sga, sgb, soa, sob, sidx):
        wid = lax.axis_index("s") * NC + lax.axis_index("c")
        base = wid * B_PER_W
        ivs = (iv0, iv1, iv2)

        d0 = pltpu.async_copy(i0_hbm.at[pl.ds(base, B_PER_W)], iv0, sidx)
        d1 = pltpu.async_copy(i1_hbm.at[pl.ds(base, B_PER_W)], iv1, sidx)
        d2 = pltpu.async_copy(i2_hbm.at[pl.ds(base, B_PER_W)], iv2, sidx)
        d0.wait()
        d1.wait()
        d2.wait()

        def g_desc(j, buf, sem, c):
            return pltpu.make_async_copy(
                y_hbm.at[ivs[j].at[pl.ds(c * GW, GW)]], buf, sem)

        def o_desc(acc, sem, c):
            return pltpu.make_async_copy(
                acc, out_hbm.at[pl.ds(base + c * GW, GW)], sem)

        def gather_start(bufs, sem, c):
            for j in range(3):
                g_desc(j, bufs[j], sem, c).start()

        def gather_wait(bufs, sem, c):
            for j in range(3):
                g_desc(j, bufs[j], sem, c).wait()

        def compute(s0, s1, s2, acc):
            @pl.loop(0, GW)
            def _(r):
                @pl.loop(0, H, step=16)
                def _(l):
                    sl = pl.ds(l, 16)
                    acc.at[r, sl][...] = (s0.at[r, sl][...]
                                          + s1.at[r, sl][...]
                                          + s2.at[r, sl][...])

        gather_start((ga0, ga1, ga2), sga, 0)

        @pl.loop(0, CHUNKS, step=2)
        def _(c):
            gather_start((gb0, gb1, gb2), sgb, c + 1)
            gather_wait((ga0, ga1, ga2), sga, c)

            @pl.when(c > 0)
            def _():
                o_desc(aca, soa, c - 2).wait()

            compute(ga0, ga1, ga2, aca)
            o_desc(aca, soa, c).start()

            @pl.when(c < CHUNKS - 2)
            def _():
                gather_start((ga0, ga1, ga2), sga, c + 2)

            gather_wait((gb0, gb1, gb2), sgb, c + 1)

            @pl.when(c > 0)
            def _():
                o_desc(acb, sob, c - 1).wait()

            compute(gb0, gb1, gb2, acb)
            o_desc(acb, sob, c + 1).start()

        o_desc(aca, soa, CHUNKS - 2).wait()
        o_desc(acb, sob, CHUNKS - 1).wait()

    return k(y_tbl, i0h, i1h, i2h)


def _sc_gather(tbl, idx):
    @functools.partial(
        pl.kernel,
        out_type=jax.ShapeDtypeStruct((PAD, H), jnp.float32),
        mesh=_sc_mesh(),
        compiler_params=_SC_PARAMS,
        scratch_types=[
            pltpu.VMEM((GW,), jnp.int32),
            pltpu.VMEM((GW, H), jnp.float32),
            pltpu.SemaphoreType.DMA,
        ],
    )
    def k(t_hbm, i_hbm, out_hbm, iv, rows, sem):
        wid = lax.axis_index("s") * NC + lax.axis_index("c")
        base = wid * B_PER_W

        @pl.loop(0, CHUNKS)
        def _(c):
            off = base + c * GW
            pltpu.sync_copy(i_hbm.at[pl.ds(off, GW)], iv)
            pltpu.async_copy(t_hbm.at[iv], rows, sem).wait()
            pltpu.sync_copy(rows, out_hbm.at[pl.ds(off, GW)])

    return k(tbl, idx)


def kernel(node_features, edge_index, current_focal_leaf, branch_child,
           time_value, is_root, W1, b1, W2, b2, Wh1, bh1, Wh2, bh2, Wh3, bh3):
    f32 = jnp.float32

    safe = jnp.where(edge_index < 0, jnp.int32(N), edge_index)
    safe = jnp.pad(safe, ((0, PAD - N), (0, 0)), constant_values=N)
    i0, i1, i2 = safe[:, 0], safe[:, 1], safe[:, 2]
    bc = jnp.pad(branch_child, (0, PAD - A))

    b1r = b1.reshape(1, H)
    b2r = b2.reshape(1, H)
    bh1r = bh1.reshape(1, H)
    bh2r = bh2.reshape(1, H)
    bh3r = bh3.reshape(1, 1)

    g_rows = PAD // BN
    g_n = -(-N // BN)
    g_a = -(-A // BN)

    wspec = lambda shape: pl.BlockSpec(shape, lambda i: (0, 0))

    y1 = pl.pallas_call(
        _mm1_body,
        grid=(g_rows,),
        in_specs=[
            pl.BlockSpec((BN, F_IN), lambda i: (i, 0)),
            pl.BlockSpec((BN, 3), lambda i: (i, 0)),
            wspec((F_IN, H)),
            wspec((1, H)),
        ],
        out_specs=pl.BlockSpec((BN, H), lambda i: (i, 0)),
        out_shape=jax.ShapeDtypeStruct((PAD, H), f32),
    )(node_features, edge_index, W1, b1r)

    agg1 = _sc_agg(y1, i0, i1, i2)

    y2 = pl.pallas_call(
        _mm2_body,
        grid=(g_rows,),
        in_specs=[
            pl.BlockSpec((BN, H), lambda i: (i, 0)),
            pl.BlockSpec((BN, H), lambda i: (i, 0)),
            pl.BlockSpec((BN, 3), lambda i: (i, 0)),
            wspec((H, H)),
            wspec((1, H)),
        ],
        out_specs=pl.BlockSpec((BN, H), lambda i: (i, 0)),
        out_shape=jax.ShapeDtypeStruct((PAD, H), f32),
    )(y1, agg1, edge_index, W2, b2r)

    agg2 = _sc_agg(y2, i0, i1, i2)

    node_emb = pl.pallas_call(
        _emb_body,
        grid=(g_n,),
        in_specs=[
            pl.BlockSpec((BN, H), lambda i: (i, 0)),
            pl.BlockSpec((BN, H), lambda i: (i, 0)),
            pl.BlockSpec((BN, 3), lambda i: (i, 0)),
        ],
        out_specs=pl.BlockSpec((BN, H), lambda i: (i, 0)),
        out_shape=jax.ShapeDtypeStruct((N, H), f32),
    )(y2, agg2, edge_index)

    h_target = _sc_gather(node_emb, bc)

    h_focal = lax.dynamic_slice(node_emb, (N - 1, 0), (1, H))

    ef, lg = pl.pallas_call(
        _mlp_body,
        grid=(g_a,),
        in_specs=[
            pl.BlockSpec((BN, H), lambda i: (i, 0)),
            pl.BlockSpec((BN, 1), lambda i: (i, 0)),
            pl.BlockSpec((BN, 1), lambda i: (i, 0)),
            wspec((1, H)),
            wspec((4 * H + 2, H)),
            wspec((1, H)),
            wspec((H, H)),
            wspec((1, H)),
            wspec((H, 1)),
            wspec((1, 1)),
        ],
        out_specs=[
            pl.BlockSpec((BN, 4 * H + 2), lambda i: (i, 0)),
            pl.BlockSpec((BN, 1), lambda i: (i, 0)),
        ],
        out_shape=[
            jax.ShapeDtypeStruct((A, 4 * H + 2), f32),
            jax.ShapeDtypeStruct((A, 1), f32),
        ],
    )(h_target, time_value.reshape(A, 1), is_root.reshape(A, 1), h_focal,
      Wh1, bh1r, Wh2, bh2r, Wh3, bh3r)

    probs = pl.pallas_call(
        _softmax_body,
        grid=(1,),
        in_specs=[pl.BlockSpec((8, A // 8), lambda i: (0, 0))],
        out_specs=pl.BlockSpec((8, A // 8), lambda i: (0, 0)),
        out_shape=jax.ShapeDtypeStruct((8, A // 8), f32),
    )(lg.reshape(8, A // 8))

    action_logits = lg.reshape(A)
    action_probs = probs.reshape(A)
    leaf_feature = jax.nn.one_hot(current_focal_leaf, F_IN, dtype=f32)
    return (action_logits, action_probs, ef, node_emb, leaf_feature)


# speedup vs baseline: 1.6641x; 1.0005x over previous
import functools

import jax
import jax.numpy as jnp
from jax import lax
from jax.experimental import pallas as pl
from jax.experimental.pallas import tpu as pltpu
from jax.experimental.pallas import tpu_sc as plsc

N = 100001
F_IN = 128
H = 64
A = 100000

NC, NS = 2, 16
NW = NC * NS
BN = 512
PAD = 100352
B_PER_W = PAD // NW
GW = 112
CHUNKS = B_PER_W // GW


def _isd_block(ei):
    mask = (ei >= 0).astype(jnp.float32)
    deg = jnp.sum(mask, axis=1, keepdims=True) + 1.0
    return lax.rsqrt(deg)


def _mm1_body(x_ref, ei_ref, w_ref, b_ref, y_ref):
    i = pl.program_id(0)
    rows = i * BN + lax.broadcasted_iota(jnp.int32, (BN, 1), 0)
    isd = _isd_block(ei_ref[...])
    xw = jnp.dot(x_ref[...], w_ref[...], preferred_element_type=jnp.float32)
    y = isd * (xw + b_ref[...])
    y_ref[...] = jnp.where(rows < N, y, 0.0)


def _mm2_body(y_ref, a_ref, ei_ref, w_ref, b_ref, o_ref):
    i = pl.program_id(0)
    rows = i * BN + lax.broadcasted_iota(jnp.int32, (BN, 1), 0)
    isd = _isd_block(ei_ref[...])
    h = jnp.maximum(isd * (y_ref[...] + a_ref[...]), 0.0)
    xw = jnp.dot(h, w_ref[...], preferred_element_type=jnp.float32)
    y2 = isd * (xw + b_ref[...])
    o_ref[...] = jnp.where(rows < N, y2, 0.0)


def _emb_body(y_ref, a_ref, ei_ref, o_ref):
    isd = _isd_block(ei_ref[...])
    o_ref[...] = isd * (y_ref[...] + a_ref[...])


def _elu(x):
    return jnp.where(x > 0, x, jnp.exp(x) - 1.0)


def _mlp_body(ht_ref, tv_ref, ir_ref, hf_ref, w1_ref, b1_ref, w2_ref, b2_ref,
              w3_ref, b3_ref, ef_ref, lg_ref):
    ht = ht_ref[...]
    hfb = jnp.broadcast_to(hf_ref[...], ht.shape)
    ad = jnp.abs(hfb - ht)
    pr = hfb * ht
    t = tv_ref[...] / jnp.float32(1.0 + 1e-8)
    ef = jnp.concatenate([hfb, ht, ad, pr, t, ir_ref[...]], axis=1)
    ef_ref[...] = ef
    z = _elu(jnp.dot(ef, w1_ref[...], preferred_element_type=jnp.float32)
             + b1_ref[...])
    z = _elu(jnp.dot(z, w2_ref[...], preferred_element_type=jnp.float32)
             + b2_ref[...])
    lg_ref[...] = (jnp.dot(z, w3_ref[...], preferred_element_type=jnp.float32)
                   + b3_ref[...])


def _softmax_body(x_ref, o_ref):
    x = x_ref[...]
    m = jnp.max(x)
    e = jnp.exp(x - m)
    o_ref[...] = e / jnp.sum(e)


def _sc_mesh():
    return plsc.VectorSubcoreMesh(core_axis_name="c", subcore_axis_name="s",
                                  num_cores=NC, num_subcores=NS)


_SC_PARAMS = pltpu.CompilerParams(use_tc_tiling_on_sc=False)


def _sc_agg(y_tbl, i0h, i1h, i2h):
    @functools.partial(
        pl.kernel,
        out_type=jax.ShapeDtypeStruct((PAD, H), jnp.float32),
        mesh=_sc_mesh(),
        compiler_params=_SC_PARAMS,
        scratch_types=[
            pltpu.VMEM((B_PER_W,), jnp.int32),
            pltpu.VMEM((B_PER_W,), jnp.int32),
            pltpu.VMEM((B_PER_W,), jnp.int32),
            pltpu.VMEM((GW, H), jnp.float32),
            pltpu.VMEM((GW, H), jnp.float32),
            pltpu.VMEM((GW, H), jnp.float32),
            pltpu.VMEM((GW, H), jnp.float32),
            pltpu.VMEM((GW, H), jnp.float32),
            pltpu.VMEM((GW, H), jnp.float32),
            pltpu.VMEM((GW, H), jnp.float32),
            pltpu.VMEM((GW, H), jnp.float32),
            pltpu.SemaphoreType.DMA,
            pltpu.SemaphoreType.DMA,
            pltpu.SemaphoreType.DMA,
            pltpu.SemaphoreType.DMA,
            pltpu.SemaphoreType.DMA,
        ],
    )
    def k(y_hbm, i0_hbm, i1_hbm, i2_hbm, out_hbm,
          iv0, iv1, iv2, ga0, ga1, ga2, gb0, gb1, gb2, aca, acb,
          sga, sgb, soa, sob, sidx):
        wid = lax.axis_index("s") * NC + lax.axis_index("c")
        base = wid * B_PER_W
        ivs = (iv0, iv1, iv2)

        d0 = pltpu.async_copy(i0_hbm.at[pl.ds(base, B_PER_W)], iv0, sidx)
        d1 = pltpu.async_copy(i1_hbm.at[pl.ds(base, B_PER_W)], iv1, sidx)
        d2 = pltpu.async_copy(i2_hbm.at[pl.ds(base, B_PER_W)], iv2, sidx)
        d0.wait()
        d1.wait()
        d2.wait()

        def g_desc(j, buf, sem, c):
            return pltpu.make_async_copy(
                y_hbm.at[ivs[j].at[pl.ds(c * GW, GW)]], buf, sem)

        def o_desc(acc, sem, c):
            return pltpu.make_async_copy(
                acc, out_hbm.at[pl.ds(base + c * GW, GW)], sem)

        def gather_start(bufs, sem, c):
            for j in range(3):
                g_desc(j, bufs[j], sem, c).start()

        def gather_wait(bufs, sem, c):
            for j in range(3):
                g_desc(j, bufs[j], sem, c).wait()

        def compute(s0, s1, s2, acc):
            @plsc.parallel_loop(0, GW, step=1, unroll=4)
            def _(r):
                for l in range(0, H, 16):
                    sl = pl.ds(l, 16)
                    acc.at[r, sl][...] = (s0.at[r, sl][...]
                                          + s1.at[r, sl][...]
                                          + s2.at[r, sl][...])

        gather_start((ga0, ga1, ga2), sga, 0)

        @pl.loop(0, CHUNKS, step=2)
        def _(c):
            gather_start((gb0, gb1, gb2), sgb, c + 1)
            gather_wait((ga0, ga1, ga2), sga, c)

            @pl.when(c > 0)
            def _():
                o_desc(aca, soa, c - 2).wait()

            compute(ga0, ga1, ga2, aca)
            o_desc(aca, soa, c).start()

            @pl.when(c < CHUNKS - 2)
            def _():
                gather_start((ga0, ga1, ga2), sga, c + 2)

            gather_wait((gb0, gb1, gb2), sgb, c + 1)

            @pl.when(c > 0)
            def _():
                o_desc(acb, sob, c - 1).wait()

            compute(gb0, gb1, gb2, acb)
            o_desc(acb, sob, c + 1).start()

        o_desc(aca, soa, CHUNKS - 2).wait()
        o_desc(acb, sob, CHUNKS - 1).wait()

    return k(y_tbl, i0h, i1h, i2h)


def _sc_gather(tbl, idx):
    @functools.partial(
        pl.kernel,
        out_type=jax.ShapeDtypeStruct((PAD, H), jnp.float32),
        mesh=_sc_mesh(),
        compiler_params=_SC_PARAMS,
        scratch_types=[
            pltpu.VMEM((GW,), jnp.int32),
            pltpu.VMEM((GW, H), jnp.float32),
            pltpu.SemaphoreType.DMA,
        ],
    )
    def k(t_hbm, i_hbm, out_hbm, iv, rows, sem):
        wid = lax.axis_index("s") * NC + lax.axis_index("c")
        base = wid * B_PER_W

        @pl.loop(0, CHUNKS)
        def _(c):
            off = base + c * GW
            pltpu.sync_copy(i_hbm.at[pl.ds(off, GW)], iv)
            pltpu.async_copy(t_hbm.at[iv], rows, sem).wait()
            pltpu.sync_copy(rows, out_hbm.at[pl.ds(off, GW)])

    return k(tbl, idx)


def kernel(node_features, edge_index, current_focal_leaf, branch_child,
           time_value, is_root, W1, b1, W2, b2, Wh1, bh1, Wh2, bh2, Wh3, bh3):
    f32 = jnp.float32

    safe = jnp.where(edge_index < 0, jnp.int32(N), edge_index)
    safe = jnp.pad(safe, ((0, PAD - N), (0, 0)), constant_values=N)
    i0, i1, i2 = safe[:, 0], safe[:, 1], safe[:, 2]
    bc = jnp.pad(branch_child, (0, PAD - A))

    b1r = b1.reshape(1, H)
    b2r = b2.reshape(1, H)
    bh1r = bh1.reshape(1, H)
    bh2r = bh2.reshape(1, H)
    bh3r = bh3.reshape(1, 1)

    g_rows = PAD // BN
    g_n = -(-N // BN)
    g_a = -(-A // BN)

    wspec = lambda shape: pl.BlockSpec(shape, lambda i: (0, 0))

    y1 = pl.pallas_call(
        _mm1_body,
        grid=(g_rows,),
        in_specs=[
            pl.BlockSpec((BN, F_IN), lambda i: (i, 0)),
            pl.BlockSpec((BN, 3), lambda i: (i, 0)),
            wspec((F_IN, H)),
            wspec((1, H)),
        ],
        out_specs=pl.BlockSpec((BN, H), lambda i: (i, 0)),
        out_shape=jax.ShapeDtypeStruct((PAD, H), f32),
    )(node_features, edge_index, W1, b1r)

    agg1 = _sc_agg(y1, i0, i1, i2)

    y2 = pl.pallas_call(
        _mm2_body,
        grid=(g_rows,),
        in_specs=[
            pl.BlockSpec((BN, H), lambda i: (i, 0)),
            pl.BlockSpec((BN, H), lambda i: (i, 0)),
            pl.BlockSpec((BN, 3), lambda i: (i, 0)),
            wspec((H, H)),
            wspec((1, H)),
        ],
        out_specs=pl.BlockSpec((BN, H), lambda i: (i, 0)),
        out_shape=jax.ShapeDtypeStruct((PAD, H), f32),
    )(y1, agg1, edge_index, W2, b2r)

    agg2 = _sc_agg(y2, i0, i1, i2)

    node_emb = pl.pallas_call(
        _emb_body,
        grid=(g_n,),
        in_specs=[
            pl.BlockSpec((BN, H), lambda i: (i, 0)),
            pl.BlockSpec((BN, H), lambda i: (i, 0)),
            pl.BlockSpec((BN, 3), lambda i: (i, 0)),
        ],
        out_specs=pl.BlockSpec((BN, H), lambda i: (i, 0)),
        out_shape=jax.ShapeDtypeStruct((N, H), f32),
    )(y2, agg2, edge_index)

    h_target = _sc_gather(node_emb, bc)

    h_focal = lax.dynamic_slice(node_emb, (N - 1, 0), (1, H))

    ef, lg = pl.pallas_call(
        _mlp_body,
        grid=(g_a,),
        in_specs=[
            pl.BlockSpec((BN, H), lambda i: (i, 0)),
            pl.BlockSpec((BN, 1), lambda i: (i, 0)),
            pl.BlockSpec((BN, 1), lambda i: (i, 0)),
            wspec((1, H)),
            wspec((4 * H + 2, H)),
            wspec((1, H)),
            wspec((H, H)),
            wspec((1, H)),
            wspec((H, 1)),
            wspec((1, 1)),
        ],
        out_specs=[
            pl.BlockSpec((BN, 4 * H + 2), lambda i: (i, 0)),
            pl.BlockSpec((BN, 1), lambda i: (i, 0)),
        ],
        out_shape=[
            jax.ShapeDtypeStruct((A, 4 * H + 2), f32),
            jax.ShapeDtypeStruct((A, 1), f32),
        ],
    )(h_target, time_value.reshape(A, 1), is_root.reshape(A, 1), h_focal,
      Wh1, bh1r, Wh2, bh2r, Wh3, bh3r)

    probs = pl.pallas_call(
        _softmax_body,
        grid=(1,),
        in_specs=[pl.BlockSpec((8, A // 8), lambda i: (0, 0))],
        out_specs=pl.BlockSpec((8, A // 8), lambda i: (0, 0)),
        out_shape=jax.ShapeDtypeStruct((8, A // 8), f32),
    )(lg.reshape(8, A // 8))

    action_logits = lg.reshape(A)
    action_probs = probs.reshape(A)
    leaf_feature = jax.nn.one_hot(current_focal_leaf, F_IN, dtype=f32)
    return (action_logits, action_probs, ef, node_emb, leaf_feature)


# speedup vs baseline: 5.2791x; 3.1723x over previous
import functools

import jax
import jax.numpy as jnp
from jax import lax
from jax.experimental import pallas as pl
from jax.experimental.pallas import tpu as pltpu
from jax.experimental.pallas import tpu_sc as plsc

N = 100001
F_IN = 128
H = 64
A = 100000

NC, NS = 2, 16
NW = NC * NS
BN = 512
PAD = 100352
B_PER_W = PAD // NW
GW = 112
CHUNKS = B_PER_W // GW


def _isd_block(ei):
    mask = (ei >= 0).astype(jnp.float32)
    deg = jnp.sum(mask, axis=1, keepdims=True) + 1.0
    return lax.rsqrt(deg)


def _mm1_body(x_ref, ei_ref, w_ref, b_ref, y_ref):
    i = pl.program_id(0)
    rows = i * BN + lax.broadcasted_iota(jnp.int32, (BN, 1), 0)
    isd = _isd_block(ei_ref[...])
    xw = jnp.dot(x_ref[...], w_ref[...], preferred_element_type=jnp.float32)
    y = isd * (xw + b_ref[...])
    y_ref[...] = jnp.where(rows < N, y, 0.0)


def _mm2_body(y_ref, a_ref, ei_ref, w_ref, b_ref, o_ref):
    i = pl.program_id(0)
    rows = i * BN + lax.broadcasted_iota(jnp.int32, (BN, 1), 0)
    isd = _isd_block(ei_ref[...])
    h = jnp.maximum(isd * (y_ref[...] + a_ref[...]), 0.0)
    xw = jnp.dot(h, w_ref[...], preferred_element_type=jnp.float32)
    y2 = isd * (xw + b_ref[...])
    o_ref[...] = jnp.where(rows < N, y2, 0.0)


def _emb_body(y_ref, a_ref, ei_ref, o_ref):
    isd = _isd_block(ei_ref[...])
    o_ref[...] = isd * (y_ref[...] + a_ref[...])


def _elu(x):
    return jnp.where(x > 0, x, jnp.exp(x) - 1.0)


def _mlp_body(ht_ref, tv_ref, ir_ref, hf_ref, w1_ref, b1_ref, w2_ref, b2_ref,
              w3_ref, b3_ref, ef_ref, lg_ref):
    ht = ht_ref[...]
    hfb = jnp.broadcast_to(hf_ref[...], ht.shape)
    ad = jnp.abs(hfb - ht)
    pr = hfb * ht
    t = tv_ref[...] / jnp.float32(1.0 + 1e-8)
    ef = jnp.concatenate([hfb, ht, ad, pr, t, ir_ref[...]], axis=1)
    ef_ref[...] = ef
    z = _elu(jnp.dot(ef, w1_ref[...], preferred_element_type=jnp.float32)
             + b1_ref[...])
    z = _elu(jnp.dot(z, w2_ref[...], preferred_element_type=jnp.float32)
             + b2_ref[...])
    lg_ref[...] = (jnp.dot(z, w3_ref[...], preferred_element_type=jnp.float32)
                   + b3_ref[...])


def _softmax_body(x_ref, o_ref):
    x = x_ref[...]
    m = jnp.max(x)
    e = jnp.exp(x - m)
    o_ref[...] = e / jnp.sum(e)


def _sc_mesh():
    return plsc.VectorSubcoreMesh(core_axis_name="c", subcore_axis_name="s",
                                  num_cores=NC, num_subcores=NS)


_SC_PARAMS = pltpu.CompilerParams(use_tc_tiling_on_sc=False)


def _sc_agg(y_tbl, i0h, i1h, i2h):
    @functools.partial(
        pl.kernel,
        out_type=jax.ShapeDtypeStruct((PAD, H), jnp.float32),
        mesh=_sc_mesh(),
        compiler_params=_SC_PARAMS,
        scratch_types=[
            pltpu.VMEM((B_PER_W,), jnp.int32),
            pltpu.VMEM((B_PER_W,), jnp.int32),
            pltpu.VMEM((B_PER_W,), jnp.int32),
            pltpu.VMEM((GW, H), jnp.float32),
            pltpu.VMEM((GW, H), jnp.float32),
            pltpu.VMEM((GW, H), jnp.float32),
            pltpu.VMEM((GW, H), jnp.float32),
            pltpu.VMEM((GW, H), jnp.float32),
            pltpu.VMEM((GW, H), jnp.float32),
            pltpu.VMEM((GW, H), jnp.float32),
            pltpu.VMEM((GW, H), jnp.float32),
            pltpu.SemaphoreType.DMA,
            pltpu.SemaphoreType.DMA,
            pltpu.SemaphoreType.DMA,
            pltpu.SemaphoreType.DMA,
            pltpu.SemaphoreType.DMA,
        ],
    )
    def k(y_hbm, i0_hbm, i1_hbm, i2_hbm, out_hbm,
          iv0, iv1, iv2, ga0, ga1, ga2, gb0, gb1, gb2, aca, acb,
          sga, sgb, soa, sob, sidx):
        wid = lax.axis_index("s") * NC + lax.axis_index("c")
        base = wid * B_PER_W
        ivs = (iv0, iv1, iv2)

        d0 = pltpu.async_copy(i0_hbm.at[pl.ds(base, B_PER_W)], iv0, sidx)
        d1 = pltpu.async_copy(i1_hbm.at[pl.ds(base, B_PER_W)], iv1, sidx)
        d2 = pltpu.async_copy(i2_hbm.at[pl.ds(base, B_PER_W)], iv2, sidx)
        d0.wait()
        d1.wait()
        d2.wait()

        def g_desc(j, buf, sem, c):
            return pltpu.make_async_copy(
                y_hbm.at[ivs[j].at[pl.ds(c * GW, GW)]], buf, sem)

        def o_desc(acc, sem, c):
            return pltpu.make_async_copy(
                acc, out_hbm.at[pl.ds(base + c * GW, GW)], sem)

        def gather_start(bufs, sem, c):
            for j in range(3):
                g_desc(j, bufs[j], sem, c).start()

        def gather_wait(bufs, sem, c):
            for j in range(3):
                g_desc(j, bufs[j], sem, c).wait()

        def compute(s0, s1, s2, acc):
            @plsc.parallel_loop(0, GW, step=1, unroll=4)
            def _(r):
                for l in range(0, H, 16):
                    sl = pl.ds(l, 16)
                    acc.at[r, sl][...] = (s0.at[r, sl][...]
                                          + s1.at[r, sl][...]
                                          + s2.at[r, sl][...])

        gather_start((ga0, ga1, ga2), sga, 0)

        @pl.loop(0, CHUNKS, step=2)
        def _(c):
            gather_start((gb0, gb1, gb2), sgb, c + 1)
            gather_wait((ga0, ga1, ga2), sga, c)

            @pl.when(c > 0)
            def _():
                o_desc(aca, soa, c - 2).wait()

            compute(ga0, ga1, ga2, aca)
            o_desc(aca, soa, c).start()

            @pl.when(c < CHUNKS - 2)
            def _():
                gather_start((ga0, ga1, ga2), sga, c + 2)

            gather_wait((gb0, gb1, gb2), sgb, c + 1)

            @pl.when(c > 0)
            def _():
                o_desc(acb, sob, c - 1).wait()

            compute(gb0, gb1, gb2, acb)
            o_desc(acb, sob, c + 1).start()

        o_desc(aca, soa, CHUNKS - 2).wait()
        o_desc(acb, sob, CHUNKS - 1).wait()

    return k(y_tbl, i0h, i1h, i2h)


def _sc_gather(tbl, idx):
    @functools.partial(
        pl.kernel,
        out_type=jax.ShapeDtypeStruct((PAD, H), jnp.float32),
        mesh=_sc_mesh(),
        compiler_params=_SC_PARAMS,
        scratch_types=[
            pltpu.VMEM((GW,), jnp.int32),
            pltpu.VMEM((GW, H), jnp.float32),
            pltpu.SemaphoreType.DMA,
        ],
    )
    def k(t_hbm, i_hbm, out_hbm, iv, rows, sem):
        wid = lax.axis_index("s") * NC + lax.axis_index("c")
        base = wid * B_PER_W

        @pl.loop(0, CHUNKS)
        def _(c):
            off = base + c * GW
            pltpu.sync_copy(i_hbm.at[pl.ds(off, GW)], iv)
            pltpu.async_copy(t_hbm.at[iv], rows, sem).wait()
            pltpu.sync_copy(rows, out_hbm.at[pl.ds(off, GW)])

    return k(tbl, idx)


def kernel(node_features, edge_index, current_focal_leaf, branch_child,
           time_value, is_root, W1, b1, W2, b2, Wh1, bh1, Wh2, bh2, Wh3, bh3):
    f32 = jnp.float32

    ei_pad = jnp.pad(edge_index, ((0, PAD - N), (0, 0)), constant_values=-1)
    rr = jnp.arange(PAD, dtype=jnp.int32)[:, None]
    cc = jnp.arange(3, dtype=jnp.int32)[None, :]
    dummy = N + ((3 * rr + cc) % (PAD - N))
    safe = jnp.where(ei_pad < 0, dummy, ei_pad)
    i0, i1, i2 = safe[:, 0], safe[:, 1], safe[:, 2]
    bc = jnp.pad(branch_child, (0, PAD - A))

    b1r = b1.reshape(1, H)
    b2r = b2.reshape(1, H)
    bh1r = bh1.reshape(1, H)
    bh2r = bh2.reshape(1, H)
    bh3r = bh3.reshape(1, 1)

    g_rows = PAD // BN
    g_n = -(-N // BN)
    g_a = -(-A // BN)

    wspec = lambda shape: pl.BlockSpec(shape, lambda i: (0, 0))

    y1 = pl.pallas_call(
        _mm1_body,
        grid=(g_rows,),
        in_specs=[
            pl.BlockSpec((BN, F_IN), lambda i: (i, 0)),
            pl.BlockSpec((BN, 3), lambda i: (i, 0)),
            wspec((F_IN, H)),
            wspec((1, H)),
        ],
        out_specs=pl.BlockSpec((BN, H), lambda i: (i, 0)),
        out_shape=jax.ShapeDtypeStruct((PAD, H), f32),
    )(node_features, edge_index, W1, b1r)

    agg1 = _sc_agg(y1, i0, i1, i2)

    y2 = pl.pallas_call(
        _mm2_body,
        grid=(g_rows,),
        in_specs=[
            pl.BlockSpec((BN, H), lambda i: (i, 0)),
            pl.BlockSpec((BN, H), lambda i: (i, 0)),
            pl.BlockSpec((BN, 3), lambda i: (i, 0)),
            wspec((H, H)),
            wspec((1, H)),
        ],
        out_specs=pl.BlockSpec((BN, H), lambda i: (i, 0)),
        out_shape=jax.ShapeDtypeStruct((PAD, H), f32),
    )(y1, agg1, edge_index, W2, b2r)

    agg2 = _sc_agg(y2, i0, i1, i2)

    node_emb = pl.pallas_call(
        _emb_body,
        grid=(g_n,),
        in_specs=[
            pl.BlockSpec((BN, H), lambda i: (i, 0)),
            pl.BlockSpec((BN, H), lambda i: (i, 0)),
            pl.BlockSpec((BN, 3), lambda i: (i, 0)),
        ],
        out_specs=pl.BlockSpec((BN, H), lambda i: (i, 0)),
        out_shape=jax.ShapeDtypeStruct((N, H), f32),
    )(y2, agg2, edge_index)

    h_target = _sc_gather(node_emb, bc)

    h_focal = lax.dynamic_slice(node_emb, (N - 1, 0), (1, H))

    ef, lg = pl.pallas_call(
        _mlp_body,
        grid=(g_a,),
        in_specs=[
            pl.BlockSpec((BN, H), lambda i: (i, 0)),
            pl.BlockSpec((BN, 1), lambda i: (i, 0)),
            pl.BlockSpec((BN, 1), lambda i: (i, 0)),
            wspec((1, H)),
            wspec((4 * H + 2, H)),
            wspec((1, H)),
            wspec((H, H)),
            wspec((1, H)),
            wspec((H, 1)),
            wspec((1, 1)),
        ],
        out_specs=[
            pl.BlockSpec((BN, 4 * H + 2), lambda i: (i, 0)),
            pl.BlockSpec((BN, 1), lambda i: (i, 0)),
        ],
        out_shape=[
            jax.ShapeDtypeStruct((A, 4 * H + 2), f32),
            jax.ShapeDtypeStruct((A, 1), f32),
        ],
    )(h_target, time_value.reshape(A, 1), is_root.reshape(A, 1), h_focal,
      Wh1, bh1r, Wh2, bh2r, Wh3, bh3r)

    probs = pl.pallas_call(
        _softmax_body,
        grid=(1,),
        in_specs=[pl.BlockSpec((8, A // 8), lambda i: (0, 0))],
        out_specs=pl.BlockSpec((8, A // 8), lambda i: (0, 0)),
        out_shape=jax.ShapeDtypeStruct((8, A // 8), f32),
    )(lg.reshape(8, A // 8))

    action_logits = lg.reshape(A)
    action_probs = probs.reshape(A)
    leaf_feature = jax.nn.one_hot(current_focal_leaf, F_IN, dtype=f32)
    return (action_logits, action_probs, ef, node_emb, leaf_feature)


# speedup vs baseline: 6.9497x; 1.3165x over previous
import functools

import jax
import jax.numpy as jnp
from jax import lax
from jax.experimental import pallas as pl
from jax.experimental.pallas import tpu as pltpu
from jax.experimental.pallas import tpu_sc as plsc

N = 100001
F_IN = 128
H = 64
A = 100000

NC, NS = 2, 16
NW = NC * NS
BN = 2048
PAD = 100352
B_PER_W = PAD // NW
GW = 112
CHUNKS = B_PER_W // GW


def _isd_block(ei):
    mask = (ei >= 0).astype(jnp.float32)
    deg = jnp.sum(mask, axis=1, keepdims=True) + 1.0
    return lax.rsqrt(deg)


def _mm1_body(x_ref, ei_ref, w_ref, b_ref, y_ref):
    i = pl.program_id(0)
    rows = i * BN + lax.broadcasted_iota(jnp.int32, (BN, 1), 0)
    isd = _isd_block(ei_ref[...])
    xw = jnp.dot(x_ref[...], w_ref[...], preferred_element_type=jnp.float32)
    y = isd * (xw + b_ref[...])
    y_ref[...] = jnp.where(rows < N, y, 0.0)


def _mm2_body(y_ref, a_ref, ei_ref, w_ref, b_ref, o_ref):
    i = pl.program_id(0)
    rows = i * BN + lax.broadcasted_iota(jnp.int32, (BN, 1), 0)
    isd = _isd_block(ei_ref[...])
    h = jnp.maximum(isd * (y_ref[...] + a_ref[...]), 0.0)
    xw = jnp.dot(h, w_ref[...], preferred_element_type=jnp.float32)
    y2 = isd * (xw + b_ref[...])
    o_ref[...] = jnp.where(rows < N, y2, 0.0)


def _emb_body(y_ref, a_ref, ei_ref, o_ref):
    isd = _isd_block(ei_ref[...])
    o_ref[...] = isd * (y_ref[...] + a_ref[...])


def _elu(x):
    return jnp.where(x > 0, x, jnp.exp(x) - 1.0)


def _mlp_body(ht_ref, tv_ref, ir_ref, hf_ref, w1_ref, b1_ref, w2_ref, b2_ref,
              w3_ref, b3_ref, ef_ref, lg_ref):
    ht = ht_ref[...]
    hf = hf_ref[...]
    hfb = jnp.broadcast_to(hf, ht.shape)
    ad = jnp.abs(hfb - ht)
    pr = hfb * ht
    t = tv_ref[...] / jnp.float32(1.0 + 1e-8)
    ir = ir_ref[...]
    ef_ref[:, 0:H] = hfb
    ef_ref[:, H:2 * H] = ht
    ef_ref[:, 2 * H:3 * H] = ad
    ef_ref[:, 3 * H:4 * H] = pr
    ef_ref[:, 4 * H:4 * H + 1] = t
    ef_ref[:, 4 * H + 1:4 * H + 2] = ir
    w1 = w1_ref[...]
    dot = functools.partial(jnp.dot, preferred_element_type=jnp.float32)
    z = (dot(hf, w1[0:H])
         + dot(ht, w1[H:2 * H])
         + dot(ad, w1[2 * H:3 * H])
         + dot(pr, w1[3 * H:4 * H])
         + t * w1[4 * H:4 * H + 1, :]
         + ir * w1[4 * H + 1:4 * H + 2, :]
         + b1_ref[...])
    z = _elu(z)
    z = _elu(dot(z, w2_ref[...]) + b2_ref[...])
    lg_ref[...] = dot(z, w3_ref[...]) + b3_ref[...]


def _softmax_body(x_ref, o_ref):
    x = x_ref[...]
    m = jnp.max(x)
    e = jnp.exp(x - m)
    o_ref[...] = e / jnp.sum(e)


def _sc_mesh():
    return plsc.VectorSubcoreMesh(core_axis_name="c", subcore_axis_name="s",
                                  num_cores=NC, num_subcores=NS)


_SC_PARAMS = pltpu.CompilerParams(use_tc_tiling_on_sc=False)


def _sc_agg(y_tbl, i0h, i1h, i2h):
    @functools.partial(
        pl.kernel,
        out_type=jax.ShapeDtypeStruct((PAD, H), jnp.float32),
        mesh=_sc_mesh(),
        compiler_params=_SC_PARAMS,
        scratch_types=[
            pltpu.VMEM((B_PER_W,), jnp.int32),
            pltpu.VMEM((B_PER_W,), jnp.int32),
            pltpu.VMEM((B_PER_W,), jnp.int32),
            pltpu.VMEM((GW, H), jnp.float32),
            pltpu.VMEM((GW, H), jnp.float32),
            pltpu.VMEM((GW, H), jnp.float32),
            pltpu.VMEM((GW, H), jnp.float32),
            pltpu.VMEM((GW, H), jnp.float32),
            pltpu.VMEM((GW, H), jnp.float32),
            pltpu.VMEM((GW, H), jnp.float32),
            pltpu.VMEM((GW, H), jnp.float32),
            pltpu.SemaphoreType.DMA,
            pltpu.SemaphoreType.DMA,
            pltpu.SemaphoreType.DMA,
            pltpu.SemaphoreType.DMA,
            pltpu.SemaphoreType.DMA,
        ],
    )
    def k(y_hbm, i0_hbm, i1_hbm, i2_hbm, out_hbm,
          iv0, iv1, iv2, ga0, ga1, ga2, gb0, gb1, gb2, aca, acb,
          sga, sgb, soa, sob, sidx):
        wid = lax.axis_index("s") * NC + lax.axis_index("c")
        base = wid * B_PER_W
        ivs = (iv0, iv1, iv2)

        d0 = pltpu.async_copy(i0_hbm.at[pl.ds(base, B_PER_W)], iv0, sidx)
        d1 = pltpu.async_copy(i1_hbm.at[pl.ds(base, B_PER_W)], iv1, sidx)
        d2 = pltpu.async_copy(i2_hbm.at[pl.ds(base, B_PER_W)], iv2, sidx)
        d0.wait()
        d1.wait()
        d2.wait()

        def g_desc(j, buf, sem, c):
            return pltpu.make_async_copy(
                y_hbm.at[ivs[j].at[pl.ds(c * GW, GW)]], buf, sem)

        def o_desc(acc, sem, c):
            return pltpu.make_async_copy(
                acc, out_hbm.at[pl.ds(base + c * GW, GW)], sem)

        def gather_start(bufs, sem, c):
            for j in range(3):
                g_desc(j, bufs[j], sem, c).start()

        def gather_wait(bufs, sem, c):
            for j in range(3):
                g_desc(j, bufs[j], sem, c).wait()

        def compute(s0, s1, s2, acc):
            @plsc.parallel_loop(0, GW, step=1, unroll=4)
            def _(r):
                for l in range(0, H, 16):
                    sl = pl.ds(l, 16)
                    acc.at[r, sl][...] = (s0.at[r, sl][...]
                                          + s1.at[r, sl][...]
                                          + s2.at[r, sl][...])

        gather_start((ga0, ga1, ga2), sga, 0)

        @pl.loop(0, CHUNKS, step=2)
        def _(c):
            gather_start((gb0, gb1, gb2), sgb, c + 1)
            gather_wait((ga0, ga1, ga2), sga, c)

            @pl.when(c > 0)
            def _():
                o_desc(aca, soa, c - 2).wait()

            compute(ga0, ga1, ga2, aca)
            o_desc(aca, soa, c).start()

            @pl.when(c < CHUNKS - 2)
            def _():
                gather_start((ga0, ga1, ga2), sga, c + 2)

            gather_wait((gb0, gb1, gb2), sgb, c + 1)

            @pl.when(c > 0)
            def _():
                o_desc(acb, sob, c - 1).wait()

            compute(gb0, gb1, gb2, acb)
            o_desc(acb, sob, c + 1).start()

        o_desc(aca, soa, CHUNKS - 2).wait()
        o_desc(acb, sob, CHUNKS - 1).wait()

    return k(y_tbl, i0h, i1h, i2h)


def _sc_gather(tbl, idx):
    @functools.partial(
        pl.kernel,
        out_type=jax.ShapeDtypeStruct((PAD, H), jnp.float32),
        mesh=_sc_mesh(),
        compiler_params=_SC_PARAMS,
        scratch_types=[
            pltpu.VMEM((GW,), jnp.int32),
            pltpu.VMEM((GW, H), jnp.float32),
            pltpu.SemaphoreType.DMA,
        ],
    )
    def k(t_hbm, i_hbm, out_hbm, iv, rows, sem):
        wid = lax.axis_index("s") * NC + lax.axis_index("c")
        base = wid * B_PER_W

        @pl.loop(0, CHUNKS)
        def _(c):
            off = base + c * GW
            pltpu.sync_copy(i_hbm.at[pl.ds(off, GW)], iv)
            pltpu.async_copy(t_hbm.at[iv], rows, sem).wait()
            pltpu.sync_copy(rows, out_hbm.at[pl.ds(off, GW)])

    return k(tbl, idx)


def kernel(node_features, edge_index, current_focal_leaf, branch_child,
           time_value, is_root, W1, b1, W2, b2, Wh1, bh1, Wh2, bh2, Wh3, bh3):
    f32 = jnp.float32

    ei_pad = jnp.pad(edge_index, ((0, PAD - N), (0, 0)), constant_values=-1)
    rr = jnp.arange(PAD, dtype=jnp.int32)[:, None]
    cc = jnp.arange(3, dtype=jnp.int32)[None, :]
    dummy = N + ((3 * rr + cc) % (PAD - N))
    safe = jnp.where(ei_pad < 0, dummy, ei_pad)
    i0, i1, i2 = safe[:, 0], safe[:, 1], safe[:, 2]
    bc = jnp.pad(branch_child, (0, PAD - A))

    b1r = b1.reshape(1, H)
    b2r = b2.reshape(1, H)
    bh1r = bh1.reshape(1, H)
    bh2r = bh2.reshape(1, H)
    bh3r = bh3.reshape(1, 1)

    g_rows = PAD // BN
    g_n = -(-N // BN)
    g_a = -(-A // BN)

    wspec = lambda shape: pl.BlockSpec(shape, lambda i: (0, 0))

    y1 = pl.pallas_call(
        _mm1_body,
        grid=(g_rows,),
        in_specs=[
            pl.BlockSpec((BN, F_IN), lambda i: (i, 0)),
            pl.BlockSpec((BN, 3), lambda i: (i, 0)),
            wspec((F_IN, H)),
            wspec((1, H)),
        ],
        out_specs=pl.BlockSpec((BN, H), lambda i: (i, 0)),
        out_shape=jax.ShapeDtypeStruct((PAD, H), f32),
    )(node_features, edge_index, W1, b1r)

    agg1 = _sc_agg(y1, i0, i1, i2)

    y2 = pl.pallas_call(
        _mm2_body,
        grid=(g_rows,),
        in_specs=[
            pl.BlockSpec((BN, H), lambda i: (i, 0)),
            pl.BlockSpec((BN, H), lambda i: (i, 0)),
            pl.BlockSpec((BN, 3), lambda i: (i, 0)),
            wspec((H, H)),
            wspec((1, H)),
        ],
        out_specs=pl.BlockSpec((BN, H), lambda i: (i, 0)),
        out_shape=jax.ShapeDtypeStruct((PAD, H), f32),
    )(y1, agg1, edge_index, W2, b2r)

    agg2 = _sc_agg(y2, i0, i1, i2)

    node_emb = pl.pallas_call(
        _emb_body,
        grid=(g_n,),
        in_specs=[
            pl.BlockSpec((BN, H), lambda i: (i, 0)),
            pl.BlockSpec((BN, H), lambda i: (i, 0)),
            pl.BlockSpec((BN, 3), lambda i: (i, 0)),
        ],
        out_specs=pl.BlockSpec((BN, H), lambda i: (i, 0)),
        out_shape=jax.ShapeDtypeStruct((N, H), f32),
    )(y2, agg2, edge_index)

    h_target = _sc_gather(node_emb, bc)

    h_focal = lax.dynamic_slice(node_emb, (N - 1, 0), (1, H))

    ef, lg = pl.pallas_call(
        _mlp_body,
        grid=(g_a,),
        in_specs=[
            pl.BlockSpec((BN, H), lambda i: (i, 0)),
            pl.BlockSpec((BN, 1), lambda i: (i, 0)),
            pl.BlockSpec((BN, 1), lambda i: (i, 0)),
            wspec((1, H)),
            wspec((4 * H + 2, H)),
            wspec((1, H)),
            wspec((H, H)),
            wspec((1, H)),
            wspec((H, 1)),
            wspec((1, 1)),
        ],
        out_specs=[
            pl.BlockSpec((BN, 4 * H + 2), lambda i: (i, 0)),
            pl.BlockSpec((BN, 1), lambda i: (i, 0)),
        ],
        out_shape=[
            jax.ShapeDtypeStruct((A, 4 * H + 2), f32),
            jax.ShapeDtypeStruct((A, 1), f32),
        ],
    )(h_target, time_value.reshape(A, 1), is_root.reshape(A, 1), h_focal,
      Wh1, bh1r, Wh2, bh2r, Wh3, bh3r)

    probs = pl.pallas_call(
        _softmax_body,
        grid=(1,),
        in_specs=[pl.BlockSpec((8, A // 8), lambda i: (0, 0))],
        out_specs=pl.BlockSpec((8, A // 8), lambda i: (0, 0)),
        out_shape=jax.ShapeDtypeStruct((8, A // 8), f32),
    )(lg.reshape(8, A // 8))

    action_logits = lg.reshape(A)
    action_probs = probs.reshape(A)
    leaf_feature = jax.nn.one_hot(current_focal_leaf, F_IN, dtype=f32)
    return (action_logits, action_probs, ef, node_emb, leaf_feature)


# speedup vs baseline: 7.5575x; 1.0874x over previous
import functools

import jax
import jax.numpy as jnp
from jax import lax
from jax.experimental import pallas as pl
from jax.experimental.pallas import tpu as pltpu
from jax.experimental.pallas import tpu_sc as plsc

N = 100001
F_IN = 128
H = 64
A = 100000

NC, NS = 2, 16
NW = NC * NS
BN = 2048
PAD = 100352
B_PER_W = PAD // NW
GW = 112
CHUNKS = B_PER_W // GW


def _isd_block(ei):
    mask = (ei >= 0).astype(jnp.float32)
    deg = jnp.sum(mask, axis=1, keepdims=True) + 1.0
    return lax.rsqrt(deg)


def _mm1_body(x_ref, ei_ref, w_ref, b_ref, y_ref):
    i = pl.program_id(0)
    rows = i * BN + lax.broadcasted_iota(jnp.int32, (BN, 1), 0)
    isd = _isd_block(ei_ref[...])
    xw = jnp.dot(x_ref[...], w_ref[...], preferred_element_type=jnp.float32)
    y = isd * (xw + b_ref[...])
    y_ref[...] = jnp.where(rows < N, y, 0.0)


def _mm2_body(y_ref, a_ref, ei_ref, w_ref, b_ref, o_ref):
    i = pl.program_id(0)
    rows = i * BN + lax.broadcasted_iota(jnp.int32, (BN, 1), 0)
    isd = _isd_block(ei_ref[...])
    h = jnp.maximum(isd * (y_ref[...] + a_ref[...]), 0.0)
    xw = jnp.dot(h, w_ref[...], preferred_element_type=jnp.float32)
    y2 = isd * (xw + b_ref[...])
    o_ref[...] = jnp.where(rows < N, y2, 0.0)


def _emb_body(y_ref, a_ref, ei_ref, o_ref):
    isd = _isd_block(ei_ref[...])
    o_ref[...] = isd * (y_ref[...] + a_ref[...])


def _elu(x):
    return jnp.where(x > 0, x, jnp.exp(x) - 1.0)


def _mlp_body(ht_ref, tv_ref, ir_ref, hf_ref, w1_ref, b1_ref, w2_ref, b2_ref,
              w3_ref, b3_ref, ef_ref, lg_ref):
    ht = ht_ref[...]
    hf = hf_ref[...]
    hfb = jnp.broadcast_to(hf, ht.shape)
    ad = jnp.abs(hfb - ht)
    pr = hfb * ht
    bn = ht.shape[0]
    t = tv_ref[...].reshape(bn, 1) / jnp.float32(1.0 + 1e-8)
    ir = ir_ref[...].reshape(bn, 1)
    ef_ref[:, 0:H] = hfb
    ef_ref[:, H:2 * H] = ht
    ef_ref[:, 2 * H:3 * H] = ad
    ef_ref[:, 3 * H:4 * H] = pr
    ef_ref[:, 4 * H:4 * H + 1] = t
    ef_ref[:, 4 * H + 1:4 * H + 2] = ir
    w1 = w1_ref[...]
    dot = functools.partial(jnp.dot, preferred_element_type=jnp.float32)
    z = (dot(hf, w1[0:H])
         + dot(ht, w1[H:2 * H])
         + dot(ad, w1[2 * H:3 * H])
         + dot(pr, w1[3 * H:4 * H])
         + t * w1[4 * H:4 * H + 1, :]
         + ir * w1[4 * H + 1:4 * H + 2, :]
         + b1_ref[...])
    z = _elu(z)
    z = _elu(dot(z, w2_ref[...]) + b2_ref[...])
    z3 = dot(z, w3_ref[...]) + b3_ref[...]
    lg_ref[...] = z3.reshape(1, bn)


def _softmax_body(x_ref, o_ref):
    x = x_ref[...]
    m = jnp.max(x)
    e = jnp.exp(x - m)
    o_ref[...] = e / jnp.sum(e)


def _sc_mesh():
    return plsc.VectorSubcoreMesh(core_axis_name="c", subcore_axis_name="s",
                                  num_cores=NC, num_subcores=NS)


_SC_PARAMS = pltpu.CompilerParams(use_tc_tiling_on_sc=False)


def _sc_agg(y_tbl, idx):
    @functools.partial(
        pl.kernel,
        out_type=jax.ShapeDtypeStruct((PAD, H), jnp.float32),
        mesh=_sc_mesh(),
        compiler_params=_SC_PARAMS,
        scratch_types=[
            pltpu.VMEM((B_PER_W,), jnp.int32),
            pltpu.VMEM((B_PER_W,), jnp.int32),
            pltpu.VMEM((B_PER_W,), jnp.int32),
            pltpu.VMEM((GW, H), jnp.float32),
            pltpu.VMEM((GW, H), jnp.float32),
            pltpu.VMEM((GW, H), jnp.float32),
            pltpu.VMEM((GW, H), jnp.float32),
            pltpu.VMEM((GW, H), jnp.float32),
            pltpu.VMEM((GW, H), jnp.float32),
            pltpu.VMEM((GW, H), jnp.float32),
            pltpu.VMEM((GW, H), jnp.float32),
            pltpu.SemaphoreType.DMA,
            pltpu.SemaphoreType.DMA,
            pltpu.SemaphoreType.DMA,
            pltpu.SemaphoreType.DMA,
            pltpu.SemaphoreType.DMA,
        ],
    )
    def k(y_hbm, i_hbm, out_hbm,
          iv0, iv1, iv2, ga0, ga1, ga2, gb0, gb1, gb2, aca, acb,
          sga, sgb, soa, sob, sidx):
        wid = lax.axis_index("s") * NC + lax.axis_index("c")
        base = wid * B_PER_W
        ivs = (iv0, iv1, iv2)

        d0 = pltpu.async_copy(i_hbm.at[0, pl.ds(base, B_PER_W)], iv0, sidx)
        d1 = pltpu.async_copy(i_hbm.at[1, pl.ds(base, B_PER_W)], iv1, sidx)
        d2 = pltpu.async_copy(i_hbm.at[2, pl.ds(base, B_PER_W)], iv2, sidx)
        d0.wait()
        d1.wait()
        d2.wait()

        def g_desc(j, buf, sem, c):
            return pltpu.make_async_copy(
                y_hbm.at[ivs[j].at[pl.ds(c * GW, GW)]], buf, sem)

        def o_desc(acc, sem, c):
            return pltpu.make_async_copy(
                acc, out_hbm.at[pl.ds(base + c * GW, GW)], sem)

        def gather_start(bufs, sem, c):
            for j in range(3):
                g_desc(j, bufs[j], sem, c).start()

        def gather_wait(bufs, sem, c):
            for j in range(3):
                g_desc(j, bufs[j], sem, c).wait()

        def compute(s0, s1, s2, acc):
            @plsc.parallel_loop(0, GW, step=1, unroll=4)
            def _(r):
                for l in range(0, H, 16):
                    sl = pl.ds(l, 16)
                    acc.at[r, sl][...] = (s0.at[r, sl][...]
                                          + s1.at[r, sl][...]
                                          + s2.at[r, sl][...])

        gather_start((ga0, ga1, ga2), sga, 0)

        @pl.loop(0, CHUNKS, step=2)
        def _(c):
            gather_start((gb0, gb1, gb2), sgb, c + 1)
            gather_wait((ga0, ga1, ga2), sga, c)

            @pl.when(c > 0)
            def _():
                o_desc(aca, soa, c - 2).wait()

            compute(ga0, ga1, ga2, aca)
            o_desc(aca, soa, c).start()

            @pl.when(c < CHUNKS - 2)
            def _():
                gather_start((ga0, ga1, ga2), sga, c + 2)

            gather_wait((gb0, gb1, gb2), sgb, c + 1)

            @pl.when(c > 0)
            def _():
                o_desc(acb, sob, c - 1).wait()

            compute(gb0, gb1, gb2, acb)
            o_desc(acb, sob, c + 1).start()

        o_desc(aca, soa, CHUNKS - 2).wait()
        o_desc(acb, sob, CHUNKS - 1).wait()

    return k(y_tbl, idx)


def _sc_gather(tbl, idx):
    @functools.partial(
        pl.kernel,
        out_type=jax.ShapeDtypeStruct((PAD, H), jnp.float32),
        mesh=_sc_mesh(),
        compiler_params=_SC_PARAMS,
        scratch_types=[
            pltpu.VMEM((GW,), jnp.int32),
            pltpu.VMEM((GW, H), jnp.float32),
            pltpu.SemaphoreType.DMA,
        ],
    )
    def k(t_hbm, i_hbm, out_hbm, iv, rows, sem):
        wid = lax.axis_index("s") * NC + lax.axis_index("c")
        base = wid * B_PER_W

        @pl.loop(0, CHUNKS)
        def _(c):
            off = base + c * GW
            pltpu.sync_copy(i_hbm.at[pl.ds(off, GW)], iv)
            pltpu.async_copy(t_hbm.at[iv], rows, sem).wait()
            pltpu.sync_copy(rows, out_hbm.at[pl.ds(off, GW)])

    return k(tbl, idx)


def kernel(node_features, edge_index, current_focal_leaf, branch_child,
           time_value, is_root, W1, b1, W2, b2, Wh1, bh1, Wh2, bh2, Wh3, bh3):
    f32 = jnp.float32

    ei_t = jnp.pad(edge_index.T, ((0, 0), (0, PAD - N)), constant_values=-1)
    rr = jnp.arange(PAD, dtype=jnp.int32)[None, :]
    cc = jnp.arange(3, dtype=jnp.int32)[:, None]
    dummy = N + ((3 * rr + cc) % (PAD - N))
    safe = jnp.where(ei_t < 0, dummy, ei_t)
    bc = jnp.pad(branch_child, (0, PAD - A))

    b1r = b1.reshape(1, H)
    b2r = b2.reshape(1, H)
    bh1r = bh1.reshape(1, H)
    bh2r = bh2.reshape(1, H)
    bh3r = bh3.reshape(1, 1)

    g_rows = PAD // BN
    g_n = -(-N // BN)
    g_a = -(-A // BN)

    wspec = lambda shape: pl.BlockSpec(shape, lambda i: (0, 0))

    y1 = pl.pallas_call(
        _mm1_body,
        grid=(g_rows,),
        in_specs=[
            pl.BlockSpec((BN, F_IN), lambda i: (i, 0)),
            pl.BlockSpec((BN, 3), lambda i: (i, 0)),
            wspec((F_IN, H)),
            wspec((1, H)),
        ],
        out_specs=pl.BlockSpec((BN, H), lambda i: (i, 0)),
        out_shape=jax.ShapeDtypeStruct((PAD, H), f32),
    )(node_features, edge_index, W1, b1r)

    agg1 = _sc_agg(y1, safe)

    y2 = pl.pallas_call(
        _mm2_body,
        grid=(g_rows,),
        in_specs=[
            pl.BlockSpec((BN, H), lambda i: (i, 0)),
            pl.BlockSpec((BN, H), lambda i: (i, 0)),
            pl.BlockSpec((BN, 3), lambda i: (i, 0)),
            wspec((H, H)),
            wspec((1, H)),
        ],
        out_specs=pl.BlockSpec((BN, H), lambda i: (i, 0)),
        out_shape=jax.ShapeDtypeStruct((PAD, H), f32),
    )(y1, agg1, edge_index, W2, b2r)

    agg2 = _sc_agg(y2, safe)

    node_emb = pl.pallas_call(
        _emb_body,
        grid=(g_n,),
        in_specs=[
            pl.BlockSpec((BN, H), lambda i: (i, 0)),
            pl.BlockSpec((BN, H), lambda i: (i, 0)),
            pl.BlockSpec((BN, 3), lambda i: (i, 0)),
        ],
        out_specs=pl.BlockSpec((BN, H), lambda i: (i, 0)),
        out_shape=jax.ShapeDtypeStruct((N, H), f32),
    )(y2, agg2, edge_index)

    h_target = _sc_gather(node_emb, bc)

    h_focal = lax.dynamic_slice(node_emb, (N - 1, 0), (1, H))

    ef, lg = pl.pallas_call(
        _mlp_body,
        grid=(g_a,),
        in_specs=[
            pl.BlockSpec((BN, H), lambda i: (i, 0)),
            pl.BlockSpec((1, BN), lambda i: (0, i)),
            pl.BlockSpec((1, BN), lambda i: (0, i)),
            wspec((1, H)),
            wspec((4 * H + 2, H)),
            wspec((1, H)),
            wspec((H, H)),
            wspec((1, H)),
            wspec((H, 1)),
            wspec((1, 1)),
        ],
        out_specs=[
            pl.BlockSpec((BN, 4 * H + 2), lambda i: (i, 0)),
            pl.BlockSpec((1, BN), lambda i: (0, i)),
        ],
        out_shape=[
            jax.ShapeDtypeStruct((A, 4 * H + 2), f32),
            jax.ShapeDtypeStruct((1, A), f32),
        ],
    )(h_target, time_value.reshape(1, A), is_root.reshape(1, A), h_focal,
      Wh1, bh1r, Wh2, bh2r, Wh3, bh3r)

    probs = pl.pallas_call(
        _softmax_body,
        grid=(1,),
        in_specs=[pl.BlockSpec((1, A), lambda i: (0, 0))],
        out_specs=pl.BlockSpec((1, A), lambda i: (0, 0)),
        out_shape=jax.ShapeDtypeStruct((1, A), f32),
    )(lg)

    action_logits = lg.reshape(A)
    action_probs = probs.reshape(A)
    leaf_feature = jax.nn.one_hot(current_focal_leaf, F_IN, dtype=f32)
    return (action_logits, action_probs, ef, node_emb, leaf_feature)


# speedup vs baseline: 7.5669x; 1.0012x over previous
import functools

import jax
import jax.numpy as jnp
from jax import lax
from jax.experimental import pallas as pl
from jax.experimental.pallas import tpu as pltpu
from jax.experimental.pallas import tpu_sc as plsc

N = 100001
F_IN = 128
H = 64
A = 100000

NC, NS = 2, 16
NW = NC * NS
BN = 2048
BA = 2000
PAD = 100352
B_PER_W = PAD // NW
GW = 112
CHUNKS = B_PER_W // GW


def _isd_block(ei):
    mask = (ei >= 0).astype(jnp.float32)
    deg = jnp.sum(mask, axis=1, keepdims=True) + 1.0
    return lax.rsqrt(deg)


def _mm1_body(x_ref, ei_ref, w_ref, b_ref, y_ref):
    i = pl.program_id(0)
    rows = i * BN + lax.broadcasted_iota(jnp.int32, (BN, 1), 0)
    isd = _isd_block(ei_ref[...])
    xw = jnp.dot(x_ref[...], w_ref[...], preferred_element_type=jnp.float32)
    y = isd * (xw + b_ref[...])
    y_ref[...] = jnp.where(rows < N, y, 0.0)


def _mm2_body(y_ref, a_ref, ei_ref, w_ref, b_ref, o_ref):
    i = pl.program_id(0)
    rows = i * BN + lax.broadcasted_iota(jnp.int32, (BN, 1), 0)
    isd = _isd_block(ei_ref[...])
    h = jnp.maximum(isd * (y_ref[...] + a_ref[...]), 0.0)
    xw = jnp.dot(h, w_ref[...], preferred_element_type=jnp.float32)
    y2 = isd * (xw + b_ref[...])
    o_ref[...] = jnp.where(rows < N, y2, 0.0)


def _emb_body(y_ref, a_ref, ei_ref, o_ref):
    isd = _isd_block(ei_ref[...])
    o_ref[...] = isd * (y_ref[...] + a_ref[...])


def _elu(x):
    return jnp.where(x > 0, x, jnp.exp(x) - 1.0)


def _mlp_body(ht_ref, tv_ref, ir_ref, hf_ref, w1_ref, b1_ref, w2_ref, b2_ref,
              w3_ref, b3_ref, ef_ref, lg_ref):
    ht = ht_ref[...]
    hf = hf_ref[...]
    hfb = jnp.broadcast_to(hf, ht.shape)
    ad = jnp.abs(hfb - ht)
    pr = hfb * ht
    bn = ht.shape[0]
    t = tv_ref[...].reshape(bn, 1) / jnp.float32(1.0 + 1e-8)
    ir = ir_ref[...].reshape(bn, 1)
    ef_ref[:, 0:H] = hfb
    ef_ref[:, H:2 * H] = ht
    ef_ref[:, 2 * H:3 * H] = ad
    ef_ref[:, 3 * H:4 * H] = pr
    ef_ref[:, 4 * H:4 * H + 1] = t
    ef_ref[:, 4 * H + 1:4 * H + 2] = ir
    w1 = w1_ref[...]
    dot = functools.partial(jnp.dot, preferred_element_type=jnp.float32)
    z = (dot(hf, w1[0:H])
         + dot(ht, w1[H:2 * H])
         + dot(ad, w1[2 * H:3 * H])
         + dot(pr, w1[3 * H:4 * H])
         + t * w1[4 * H:4 * H + 1, :]
         + ir * w1[4 * H + 1:4 * H + 2, :]
         + b1_ref[...])
    z = _elu(z)
    z = _elu(dot(z, w2_ref[...]) + b2_ref[...])
    z3 = dot(z, w3_ref[...]) + b3_ref[...]
    lg_ref[...] = z3.reshape(1, 1, bn)


def _softmax_body(x_ref, o_ref):
    x = x_ref[...]
    m = jnp.max(x)
    e = jnp.exp(x - m)
    o_ref[...] = e / jnp.sum(e)


def _sc_mesh():
    return plsc.VectorSubcoreMesh(core_axis_name="c", subcore_axis_name="s",
                                  num_cores=NC, num_subcores=NS)


_SC_PARAMS = pltpu.CompilerParams(use_tc_tiling_on_sc=False)


def _sc_agg(y_tbl, idx):
    @functools.partial(
        pl.kernel,
        out_type=jax.ShapeDtypeStruct((PAD, H), jnp.float32),
        mesh=_sc_mesh(),
        compiler_params=_SC_PARAMS,
        scratch_types=[
            pltpu.VMEM((B_PER_W,), jnp.int32),
            pltpu.VMEM((B_PER_W,), jnp.int32),
            pltpu.VMEM((B_PER_W,), jnp.int32),
            pltpu.VMEM((GW, H), jnp.float32),
            pltpu.VMEM((GW, H), jnp.float32),
            pltpu.VMEM((GW, H), jnp.float32),
            pltpu.VMEM((GW, H), jnp.float32),
            pltpu.VMEM((GW, H), jnp.float32),
            pltpu.VMEM((GW, H), jnp.float32),
            pltpu.VMEM((GW, H), jnp.float32),
            pltpu.VMEM((GW, H), jnp.float32),
            pltpu.SemaphoreType.DMA,
            pltpu.SemaphoreType.DMA,
            pltpu.SemaphoreType.DMA,
            pltpu.SemaphoreType.DMA,
            pltpu.SemaphoreType.DMA,
        ],
    )
    def k(y_hbm, i_hbm, out_hbm,
          iv0, iv1, iv2, ga0, ga1, ga2, gb0, gb1, gb2, aca, acb,
          sga, sgb, soa, sob, sidx):
        wid = lax.axis_index("s") * NC + lax.axis_index("c")
        base = wid * B_PER_W
        ivs = (iv0, iv1, iv2)

        d0 = pltpu.async_copy(i_hbm.at[0, pl.ds(base, B_PER_W)], iv0, sidx)
        d1 = pltpu.async_copy(i_hbm.at[1, pl.ds(base, B_PER_W)], iv1, sidx)
        d2 = pltpu.async_copy(i_hbm.at[2, pl.ds(base, B_PER_W)], iv2, sidx)
        d0.wait()
        d1.wait()
        d2.wait()

        def g_desc(j, buf, sem, c):
            return pltpu.make_async_copy(
                y_hbm.at[ivs[j].at[pl.ds(c * GW, GW)]], buf, sem)

        def o_desc(acc, sem, c):
            return pltpu.make_async_copy(
                acc, out_hbm.at[pl.ds(base + c * GW, GW)], sem)

        def gather_start(bufs, sem, c):
            for j in range(3):
                g_desc(j, bufs[j], sem, c).start()

        def gather_wait(bufs, sem, c):
            for j in range(3):
                g_desc(j, bufs[j], sem, c).wait()

        def compute(s0, s1, s2, acc):
            @plsc.parallel_loop(0, GW, step=1, unroll=4)
            def _(r):
                for l in range(0, H, 16):
                    sl = pl.ds(l, 16)
                    acc.at[r, sl][...] = (s0.at[r, sl][...]
                                          + s1.at[r, sl][...]
                                          + s2.at[r, sl][...])

        gather_start((ga0, ga1, ga2), sga, 0)

        @pl.loop(0, CHUNKS, step=2)
        def _(c):
            gather_start((gb0, gb1, gb2), sgb, c + 1)
            gather_wait((ga0, ga1, ga2), sga, c)

            @pl.when(c > 0)
            def _():
                o_desc(aca, soa, c - 2).wait()

            compute(ga0, ga1, ga2, aca)
            o_desc(aca, soa, c).start()

            @pl.when(c < CHUNKS - 2)
            def _():
                gather_start((ga0, ga1, ga2), sga, c + 2)

            gather_wait((gb0, gb1, gb2), sgb, c + 1)

            @pl.when(c > 0)
            def _():
                o_desc(acb, sob, c - 1).wait()

            compute(gb0, gb1, gb2, acb)
            o_desc(acb, sob, c + 1).start()

        o_desc(aca, soa, CHUNKS - 2).wait()
        o_desc(acb, sob, CHUNKS - 1).wait()

    return k(y_tbl, idx)


def _sc_gather(tbl, idx):
    @functools.partial(
        pl.kernel,
        out_type=jax.ShapeDtypeStruct((PAD, H), jnp.float32),
        mesh=_sc_mesh(),
        compiler_params=_SC_PARAMS,
        scratch_types=[
            pltpu.VMEM((GW,), jnp.int32),
            pltpu.VMEM((GW, H), jnp.float32),
            pltpu.SemaphoreType.DMA,
        ],
    )
    def k(t_hbm, i_hbm, out_hbm, iv, rows, sem):
        wid = lax.axis_index("s") * NC + lax.axis_index("c")
        base = wid * B_PER_W

        @pl.loop(0, CHUNKS)
        def _(c):
            off = base + c * GW
            pltpu.sync_copy(i_hbm.at[pl.ds(off, GW)], iv)
            pltpu.async_copy(t_hbm.at[iv], rows, sem).wait()
            pltpu.sync_copy(rows, out_hbm.at[pl.ds(off, GW)])

    return k(tbl, idx)


def kernel(node_features, edge_index, current_focal_leaf, branch_child,
           time_value, is_root, W1, b1, W2, b2, Wh1, bh1, Wh2, bh2, Wh3, bh3):
    f32 = jnp.float32

    ei_t = jnp.pad(edge_index.T, ((0, 0), (0, PAD - N)), constant_values=-1)
    rr = jnp.arange(PAD, dtype=jnp.int32)[None, :]
    cc = jnp.arange(3, dtype=jnp.int32)[:, None]
    dummy = N + ((3 * rr + cc) % (PAD - N))
    safe = jnp.where(ei_t < 0, dummy, ei_t)
    bc = jnp.pad(branch_child, (0, PAD - A))

    b1r = b1.reshape(1, H)
    b2r = b2.reshape(1, H)
    bh1r = bh1.reshape(1, H)
    bh2r = bh2.reshape(1, H)
    bh3r = bh3.reshape(1, 1)

    g_rows = PAD // BN
    g_n = -(-N // BN)
    g_a = A // BA

    wspec = lambda shape: pl.BlockSpec(shape, lambda i: (0, 0))

    y1 = pl.pallas_call(
        _mm1_body,
        grid=(g_rows,),
        in_specs=[
            pl.BlockSpec((BN, F_IN), lambda i: (i, 0)),
            pl.BlockSpec((BN, 3), lambda i: (i, 0)),
            wspec((F_IN, H)),
            wspec((1, H)),
        ],
        out_specs=pl.BlockSpec((BN, H), lambda i: (i, 0)),
        out_shape=jax.ShapeDtypeStruct((PAD, H), f32),
    )(node_features, edge_index, W1, b1r)

    agg1 = _sc_agg(y1, safe)

    y2 = pl.pallas_call(
        _mm2_body,
        grid=(g_rows,),
        in_specs=[
            pl.BlockSpec((BN, H), lambda i: (i, 0)),
            pl.BlockSpec((BN, H), lambda i: (i, 0)),
            pl.BlockSpec((BN, 3), lambda i: (i, 0)),
            wspec((H, H)),
            wspec((1, H)),
        ],
        out_specs=pl.BlockSpec((BN, H), lambda i: (i, 0)),
        out_shape=jax.ShapeDtypeStruct((PAD, H), f32),
    )(y1, agg1, edge_index, W2, b2r)

    agg2 = _sc_agg(y2, safe)

    node_emb = pl.pallas_call(
        _emb_body,
        grid=(g_n,),
        in_specs=[
            pl.BlockSpec((BN, H), lambda i: (i, 0)),
            pl.BlockSpec((BN, H), lambda i: (i, 0)),
            pl.BlockSpec((BN, 3), lambda i: (i, 0)),
        ],
        out_specs=pl.BlockSpec((BN, H), lambda i: (i, 0)),
        out_shape=jax.ShapeDtypeStruct((N, H), f32),
    )(y2, agg2, edge_index)

    h_target = _sc_gather(node_emb, bc)

    h_focal = lax.dynamic_slice(node_emb, (N - 1, 0), (1, H))

    ef, lg = pl.pallas_call(
        _mlp_body,
        grid=(g_a,),
        in_specs=[
            pl.BlockSpec((BA, H), lambda i: (i, 0)),
            pl.BlockSpec((1, 1, BA), lambda i: (i, 0, 0)),
            pl.BlockSpec((1, 1, BA), lambda i: (i, 0, 0)),
            wspec((1, H)),
            wspec((4 * H + 2, H)),
            wspec((1, H)),
            wspec((H, H)),
            wspec((1, H)),
            wspec((H, 1)),
            wspec((1, 1)),
        ],
        out_specs=[
            pl.BlockSpec((BA, 4 * H + 2), lambda i: (i, 0)),
            pl.BlockSpec((1, 1, BA), lambda i: (i, 0, 0)),
        ],
        out_shape=[
            jax.ShapeDtypeStruct((A, 4 * H + 2), f32),
            jax.ShapeDtypeStruct((g_a, 1, BA), f32),
        ],
    )(h_target, time_value.reshape(g_a, 1, BA), is_root.reshape(g_a, 1, BA),
      h_focal, Wh1, bh1r, Wh2, bh2r, Wh3, bh3r)

    probs = pl.pallas_call(
        _softmax_body,
        grid=(1,),
        in_specs=[pl.BlockSpec((g_a, 1, BA), lambda i: (0, 0, 0))],
        out_specs=pl.BlockSpec((g_a, 1, BA), lambda i: (0, 0, 0)),
        out_shape=jax.ShapeDtypeStruct((g_a, 1, BA), f32),
    )(lg)

    action_logits = lg.reshape(A)
    action_probs = probs.reshape(A)
    leaf_feature = jax.nn.one_hot(current_focal_leaf, F_IN, dtype=f32)
    return (action_logits, action_probs, ef, node_emb, leaf_feature)


# speedup vs baseline: 8.5244x; 1.1265x over previous
import functools

import jax
import jax.numpy as jnp
from jax import lax
from jax.experimental import pallas as pl
from jax.experimental.pallas import tpu as pltpu
from jax.experimental.pallas import tpu_sc as plsc

N = 100001
F_IN = 128
H = 64
A = 100000

NC, NS = 2, 16
NW = NC * NS
BN = 2048
BA = 2000
PAD = 100352
B_PER_W = PAD // NW
GW = 56
CHUNKS = B_PER_W // GW


def _isd_block(ei):
    mask = (ei >= 0).astype(jnp.float32)
    deg = jnp.sum(mask, axis=1, keepdims=True) + 1.0
    return lax.rsqrt(deg)


def _mm1_body(x_ref, ei_ref, w_ref, b_ref, y_ref):
    i = pl.program_id(0)
    rows = i * BN + lax.broadcasted_iota(jnp.int32, (BN, 1), 0)
    isd = _isd_block(ei_ref[...])
    xw = jnp.dot(x_ref[...], w_ref[...], preferred_element_type=jnp.float32)
    y = isd * (xw + b_ref[...])
    y = jnp.concatenate([y, jnp.zeros_like(y)], axis=1)
    y_ref[...] = jnp.where(rows < N, y, 0.0)


def _mm2_body(y_ref, a_ref, ei_ref, w_ref, b_ref, o_ref):
    i = pl.program_id(0)
    rows = i * BN + lax.broadcasted_iota(jnp.int32, (BN, 1), 0)
    isd = _isd_block(ei_ref[...])
    h = jnp.maximum(isd * (y_ref[:, :H] + a_ref[:, :H]), 0.0)
    xw = jnp.dot(h, w_ref[...], preferred_element_type=jnp.float32)
    y2 = isd * (xw + b_ref[...])
    y2 = jnp.concatenate([y2, jnp.zeros_like(y2)], axis=1)
    o_ref[...] = jnp.where(rows < N, y2, 0.0)


def _emb_body(y_ref, a_ref, ei_ref, o_ref, o128_ref):
    isd = _isd_block(ei_ref[...])
    e = isd * (y_ref[:, :H] + a_ref[:, :H])
    o_ref[...] = e
    o128_ref[...] = jnp.concatenate([e, jnp.zeros_like(e)], axis=1)


def _elu(x):
    return jnp.where(x > 0, x, jnp.exp(x) - 1.0)


def _mlp_body(ht_ref, tv_ref, ir_ref, hf_ref, w1_ref, b1_ref, w2_ref, b2_ref,
              w3_ref, b3_ref, ef_ref, lg_ref):
    ht = ht_ref[:, :H]
    hf = hf_ref[...]
    hfb = jnp.broadcast_to(hf, ht.shape)
    ad = jnp.abs(hfb - ht)
    pr = hfb * ht
    bn = ht.shape[0]
    t = tv_ref[...].reshape(bn, 1) / jnp.float32(1.0 + 1e-8)
    ir = ir_ref[...].reshape(bn, 1)
    ef_ref[:, 0:H] = hfb
    ef_ref[:, H:2 * H] = ht
    ef_ref[:, 2 * H:3 * H] = ad
    ef_ref[:, 3 * H:4 * H] = pr
    ef_ref[:, 4 * H:4 * H + 1] = t
    ef_ref[:, 4 * H + 1:4 * H + 2] = ir
    w1 = w1_ref[...]
    dot = functools.partial(jnp.dot, preferred_element_type=jnp.float32)
    z = (dot(hf, w1[0:H])
         + dot(ht, w1[H:2 * H])
         + dot(ad, w1[2 * H:3 * H])
         + dot(pr, w1[3 * H:4 * H])
         + t * w1[4 * H:4 * H + 1, :]
         + ir * w1[4 * H + 1:4 * H + 2, :]
         + b1_ref[...])
    z = _elu(z)
    z = _elu(dot(z, w2_ref[...]) + b2_ref[...])
    z3 = dot(z, w3_ref[...]) + b3_ref[...]
    lg_ref[...] = z3.reshape(1, 1, bn)


def _softmax_body(x_ref, o_ref):
    x = x_ref[...]
    m = jnp.max(x)
    e = jnp.exp(x - m)
    o_ref[...] = e / jnp.sum(e)


def _sc_mesh():
    return plsc.VectorSubcoreMesh(core_axis_name="c", subcore_axis_name="s",
                                  num_cores=NC, num_subcores=NS)


_SC_PARAMS = pltpu.CompilerParams(use_tc_tiling_on_sc=False)


def _sc_agg(y_tbl, idx):
    @functools.partial(
        pl.kernel,
        out_type=jax.ShapeDtypeStruct((PAD, 2 * H), jnp.float32),
        mesh=_sc_mesh(),
        compiler_params=_SC_PARAMS,
        scratch_types=[
            pltpu.VMEM((B_PER_W,), jnp.int32),
            pltpu.VMEM((B_PER_W,), jnp.int32),
            pltpu.VMEM((B_PER_W,), jnp.int32),
            pltpu.VMEM((GW, 2 * H), jnp.float32),
            pltpu.VMEM((GW, 2 * H), jnp.float32),
            pltpu.VMEM((GW, 2 * H), jnp.float32),
            pltpu.VMEM((GW, 2 * H), jnp.float32),
            pltpu.VMEM((GW, 2 * H), jnp.float32),
            pltpu.VMEM((GW, 2 * H), jnp.float32),
            pltpu.VMEM((GW, 2 * H), jnp.float32),
            pltpu.VMEM((GW, 2 * H), jnp.float32),
            pltpu.SemaphoreType.DMA,
            pltpu.SemaphoreType.DMA,
            pltpu.SemaphoreType.DMA,
            pltpu.SemaphoreType.DMA,
            pltpu.SemaphoreType.DMA,
        ],
    )
    def k(y_hbm, i_hbm, out_hbm,
          iv0, iv1, iv2, ga0, ga1, ga2, gb0, gb1, gb2, aca, acb,
          sga, sgb, soa, sob, sidx):
        wid = lax.axis_index("s") * NC + lax.axis_index("c")
        base = wid * B_PER_W
        ivs = (iv0, iv1, iv2)

        d0 = pltpu.async_copy(i_hbm.at[0, pl.ds(base, B_PER_W)], iv0, sidx)
        d1 = pltpu.async_copy(i_hbm.at[1, pl.ds(base, B_PER_W)], iv1, sidx)
        d2 = pltpu.async_copy(i_hbm.at[2, pl.ds(base, B_PER_W)], iv2, sidx)
        d0.wait()
        d1.wait()
        d2.wait()

        def g_desc(j, buf, sem, c):
            return pltpu.make_async_copy(
                y_hbm.at[ivs[j].at[pl.ds(c * GW, GW)]], buf, sem)

        def o_desc(acc, sem, c):
            return pltpu.make_async_copy(
                acc, out_hbm.at[pl.ds(base + c * GW, GW)], sem)

        def gather_start(bufs, sem, c):
            for j in range(3):
                g_desc(j, bufs[j], sem, c).start()

        def gather_wait(bufs, sem, c):
            for j in range(3):
                g_desc(j, bufs[j], sem, c).wait()

        def compute(s0, s1, s2, acc):
            @plsc.parallel_loop(0, GW, step=1, unroll=4)
            def _(r):
                for l in range(0, 2 * H, 16):
                    sl = pl.ds(l, 16)
                    acc.at[r, sl][...] = (s0.at[r, sl][...]
                                          + s1.at[r, sl][...]
                                          + s2.at[r, sl][...])

        gather_start((ga0, ga1, ga2), sga, 0)

        @pl.loop(0, CHUNKS, step=2)
        def _(c):
            gather_start((gb0, gb1, gb2), sgb, c + 1)
            gather_wait((ga0, ga1, ga2), sga, c)

            @pl.when(c > 0)
            def _():
                o_desc(aca, soa, c - 2).wait()

            compute(ga0, ga1, ga2, aca)
            o_desc(aca, soa, c).start()

            @pl.when(c < CHUNKS - 2)
            def _():
                gather_start((ga0, ga1, ga2), sga, c + 2)

            gather_wait((gb0, gb1, gb2), sgb, c + 1)

            @pl.when(c > 0)
            def _():
                o_desc(acb, sob, c - 1).wait()

            compute(gb0, gb1, gb2, acb)
            o_desc(acb, sob, c + 1).start()

        o_desc(aca, soa, CHUNKS - 2).wait()
        o_desc(acb, sob, CHUNKS - 1).wait()

    return k(y_tbl, idx)


def _sc_gather(tbl, idx):
    @functools.partial(
        pl.kernel,
        out_type=jax.ShapeDtypeStruct((PAD, 2 * H), jnp.float32),
        mesh=_sc_mesh(),
        compiler_params=_SC_PARAMS,
        scratch_types=[
            pltpu.VMEM((GW,), jnp.int32),
            pltpu.VMEM((GW, 2 * H), jnp.float32),
            pltpu.SemaphoreType.DMA,
        ],
    )
    def k(t_hbm, i_hbm, out_hbm, iv, rows, sem):
        wid = lax.axis_index("s") * NC + lax.axis_index("c")
        base = wid * B_PER_W

        @pl.loop(0, CHUNKS)
        def _(c):
            off = base + c * GW
            pltpu.sync_copy(i_hbm.at[pl.ds(off, GW)], iv)
            pltpu.async_copy(t_hbm.at[iv], rows, sem).wait()
            pltpu.sync_copy(rows, out_hbm.at[pl.ds(off, GW)])

    return k(tbl, idx)


def kernel(node_features, edge_index, current_focal_leaf, branch_child,
           time_value, is_root, W1, b1, W2, b2, Wh1, bh1, Wh2, bh2, Wh3, bh3):
    f32 = jnp.float32

    ei_t = jnp.pad(edge_index.T, ((0, 0), (0, PAD - N)), constant_values=-1)
    rr = jnp.arange(PAD, dtype=jnp.int32)[None, :]
    cc = jnp.arange(3, dtype=jnp.int32)[:, None]
    dummy = N + ((3 * rr + cc) % (PAD - N))
    safe = jnp.where(ei_t < 0, dummy, ei_t)
    bc = jnp.pad(branch_child, (0, PAD - A))

    b1r = b1.reshape(1, H)
    b2r = b2.reshape(1, H)
    bh1r = bh1.reshape(1, H)
    bh2r = bh2.reshape(1, H)
    bh3r = bh3.reshape(1, 1)

    g_rows = PAD // BN
    g_n = -(-N // BN)
    g_a = A // BA

    wspec = lambda shape: pl.BlockSpec(shape, lambda i: (0, 0))

    y1 = pl.pallas_call(
        _mm1_body,
        grid=(g_rows,),
        in_specs=[
            pl.BlockSpec((BN, F_IN), lambda i: (i, 0)),
            pl.BlockSpec((BN, 3), lambda i: (i, 0)),
            wspec((F_IN, H)),
            wspec((1, H)),
        ],
        out_specs=pl.BlockSpec((BN, 2 * H), lambda i: (i, 0)),
        out_shape=jax.ShapeDtypeStruct((PAD, 2 * H), f32),
    )(node_features, edge_index, W1, b1r)

    agg1 = _sc_agg(y1, safe)

    y2 = pl.pallas_call(
        _mm2_body,
        grid=(g_rows,),
        in_specs=[
            pl.BlockSpec((BN, 2 * H), lambda i: (i, 0)),
            pl.BlockSpec((BN, 2 * H), lambda i: (i, 0)),
            pl.BlockSpec((BN, 3), lambda i: (i, 0)),
            wspec((H, H)),
            wspec((1, H)),
        ],
        out_specs=pl.BlockSpec((BN, 2 * H), lambda i: (i, 0)),
        out_shape=jax.ShapeDtypeStruct((PAD, 2 * H), f32),
    )(y1, agg1, edge_index, W2, b2r)

    agg2 = _sc_agg(y2, safe)

    node_emb, ne128 = pl.pallas_call(
        _emb_body,
        grid=(g_n,),
        in_specs=[
            pl.BlockSpec((BN, 2 * H), lambda i: (i, 0)),
            pl.BlockSpec((BN, 2 * H), lambda i: (i, 0)),
            pl.BlockSpec((BN, 3), lambda i: (i, 0)),
        ],
        out_specs=[
            pl.BlockSpec((BN, H), lambda i: (i, 0)),
            pl.BlockSpec((BN, 2 * H), lambda i: (i, 0)),
        ],
        out_shape=[
            jax.ShapeDtypeStruct((N, H), f32),
            jax.ShapeDtypeStruct((PAD, 2 * H), f32),
        ],
    )(y2, agg2, edge_index)

    h_target = _sc_gather(ne128, bc)

    h_focal = lax.dynamic_slice(node_emb, (N - 1, 0), (1, H))

    ef, lg = pl.pallas_call(
        _mlp_body,
        grid=(g_a,),
        in_specs=[
            pl.BlockSpec((BA, 2 * H), lambda i: (i, 0)),
            pl.BlockSpec((1, 1, BA), lambda i: (i, 0, 0)),
            pl.BlockSpec((1, 1, BA), lambda i: (i, 0, 0)),
            wspec((1, H)),
            wspec((4 * H + 2, H)),
            wspec((1, H)),
            wspec((H, H)),
            wspec((1, H)),
            wspec((H, 1)),
            wspec((1, 1)),
        ],
        out_specs=[
            pl.BlockSpec((BA, 4 * H + 2), lambda i: (i, 0)),
            pl.BlockSpec((1, 1, BA), lambda i: (i, 0, 0)),
        ],
        out_shape=[
            jax.ShapeDtypeStruct((A, 4 * H + 2), f32),
            jax.ShapeDtypeStruct((g_a, 1, BA), f32),
        ],
    )(h_target, time_value.reshape(g_a, 1, BA), is_root.reshape(g_a, 1, BA),
      h_focal, Wh1, bh1r, Wh2, bh2r, Wh3, bh3r)

    probs = pl.pallas_call(
        _softmax_body,
        grid=(1,),
        in_specs=[pl.BlockSpec((g_a, 1, BA), lambda i: (0, 0, 0))],
        out_specs=pl.BlockSpec((g_a, 1, BA), lambda i: (0, 0, 0)),
        out_shape=jax.ShapeDtypeStruct((g_a, 1, BA), f32),
    )(lg)

    action_logits = lg.reshape(A)
    action_probs = probs.reshape(A)
    leaf_feature = jax.nn.one_hot(current_focal_leaf, F_IN, dtype=f32)
    return (action_logits, action_probs, ef, node_emb, leaf_feature)
